# Initial kernel scaffold; baseline (speedup 1.0000x reference)
import functools

import jax
import jax.numpy as jnp
import numpy as np
from jax import lax
from jax.experimental import pallas as pl
from jax.experimental.pallas import tpu as pltpu
from jax.experimental.pallas import tpu_sc as plsc

N = 50000
E = 800000
D = 64
H = 4
C = 16
L = 4
NEG = 0.2
_ATOM_DIMS = [119, 4, 12, 12, 10, 6, 6, 2, 2]
_BOND_DIMS = [5, 6, 2]
_AOFF = np.concatenate([[0], np.cumsum(_ATOM_DIMS)[:-1]]).astype(np.int32)
_BOFF = np.concatenate([[0], np.cumsum(_BOND_DIMS)[:-1]]).astype(np.int32)
_AV = int(sum(_ATOM_DIMS))
_BV = int(sum(_BOND_DIMS))

NP_ = 50048
EP_ = 819200
BN = 6256
BE = 8192
NC, NS = 2, 16
HALF = NP_ // NC
TRASH = HALF

_R = np.zeros((H, D), np.float32)
for h in range(H):
    _R[h, h * C:(h + 1) * C] = 1.0
_RT = _R.T.copy()
_P = np.zeros((3, H, 16), np.float32)
for j in range(3):
    for h in range(H):
        _P[j, h, j * H + h] = 1.0
_S = np.zeros((3, 16, H), np.float32)
for j in range(3):
    for h in range(H):
        _S[j, j * H + h, h] = 1.0


def _full(shape):
    nd = len(shape)
    return pl.BlockSpec(shape, lambda i, _n=nd: (0,) * _n)


def _enc_body(nv, offs, nvalid, blk, x_ref, tab_ref, out_ref):
    i = pl.program_id(0)
    xb = x_ref[...]
    oh = jnp.zeros((blk, nv), jnp.float32)
    iot = lax.broadcasted_iota(jnp.int32, (blk, nv), 1)
    for k in range(len(offs)):
        col = lax.slice(xb, (0, k), (blk, k + 1)) + np.int32(offs[k])
        oh = oh + (col == iot).astype(jnp.float32)
    node = jnp.dot(oh, tab_ref[...], preferred_element_type=jnp.float32)
    rid = i * blk + lax.broadcasted_iota(jnp.int32, (blk, 1), 0)
    out_ref[...] = jnp.where(rid < nvalid, node, 0.0)


def _encode(x_pad, table, nvals, offs, nvalid, rows, blk):
    body = functools.partial(_enc_body, nvals, offs, nvalid, blk)
    return pl.pallas_call(
        body,
        grid=(rows // blk,),
        in_specs=[pl.BlockSpec((blk, x_pad.shape[1]), lambda i: (i, 0)),
                  _full(table.shape)],
        out_specs=pl.BlockSpec((blk, D), lambda i: (i, 0)),
        out_shape=jax.ShapeDtypeStruct((rows, D), jnp.float32),
    )(x_pad, table)


def _attn_pass_body(e_ref, we_ref, ae_ref, esum_ref):
    i = pl.program_id(0)
    eb = e_ref[...]
    ae_ref[...] = jnp.dot(eb, we_ref[...], preferred_element_type=jnp.float32)
    ones8 = jnp.ones((8, BE), jnp.float32)
    part = jnp.dot(ones8, eb, preferred_element_type=jnp.float32)

    @pl.when(i == 0)
    def _():
        esum_ref[...] = jnp.zeros_like(esum_ref)

    esum_ref[...] += part


def _edge_attn_pass(edge, we):
    return pl.pallas_call(
        _attn_pass_body,
        grid=(EP_ // BE,),
        in_specs=[pl.BlockSpec((BE, D), lambda i: (i, 0)), _full((D, H))],
        out_specs=[pl.BlockSpec((BE, H), lambda i: (i, 0)),
                   pl.BlockSpec((8, D), lambda i: (0, 0))],
        out_shape=[jax.ShapeDtypeStruct((EP_, H), jnp.float32),
                   jax.ShapeDtypeStruct((8, D), jnp.float32)],
    )(edge, we)


def _node_stage_body(node_ref, wl_ref, asf_ref, adf_ref, we_ref, esum_ref,
                     rt_ref, p_ref, xh_ref, tbl_ref):
    xh = jnp.dot(node_ref[...], wl_ref[...], preferred_element_type=jnp.float32)
    rt = rt_ref[...]
    a_s = jnp.dot(xh * asf_ref[...], rt, preferred_element_type=jnp.float32)
    a_d = jnp.dot(xh * adf_ref[...], rt, preferred_element_type=jnp.float32)
    e_mean = jnp.sum(esum_ref[...], 0, keepdims=True) * np.float32(1.0 / (8 * E))
    a_el = jnp.dot(e_mean, we_ref[...], preferred_element_type=jnp.float32)
    pre = a_s + a_d + a_el
    sub = jnp.where(pre >= 0, pre, pre * np.float32(NEG))
    tbl = (jnp.dot(a_s, p_ref[0], preferred_element_type=jnp.float32)
           + jnp.dot(a_d, p_ref[1], preferred_element_type=jnp.float32)
           + jnp.dot(sub, p_ref[2], preferred_element_type=jnp.float32))
    xh_ref[...] = xh
    tbl_ref[...] = tbl


def _node_stage(node, w_lin, asf, adf, we, esum8):
    return pl.pallas_call(
        _node_stage_body,
        grid=(NP_ // BN,),
        in_specs=[pl.BlockSpec((BN, D), lambda i: (i, 0)), _full((D, D)),
                  _full((1, D)), _full((1, D)), _full((D, H)), _full((8, D)),
                  _full((D, H)), _full((3, H, 16))],
        out_specs=[pl.BlockSpec((BN, D), lambda i: (i, 0)),
                   pl.BlockSpec((BN, 16), lambda i: (i, 0))],
        out_shape=[jax.ShapeDtypeStruct((NP_, D), jnp.float32),
                   jax.ShapeDtypeStruct((NP_, 16), jnp.float32)],
    )(node, w_lin, asf, adf, we, esum8, jnp.asarray(_RT), jnp.asarray(_P))


def _edge_ex_body(p1_ref, p2_ref, ae_ref, gxh_ref, s_ref, p_ref, r_ref,
                  ex16_ref, contrib_ref):
    p1 = p1_ref[...]
    p2 = p2_ref[...]
    a = (jnp.dot(p1, s_ref[0], preferred_element_type=jnp.float32)
         + jnp.dot(p2, s_ref[1], preferred_element_type=jnp.float32)
         + ae_ref[...])
    a = jnp.where(a >= 0, a, a * np.float32(NEG))
    sub = jnp.dot(p2, s_ref[2], preferred_element_type=jnp.float32)
    ex = jnp.exp(a - sub)
    ex16_ref[...] = jnp.dot(ex, p_ref[0], preferred_element_type=jnp.float32)
    contrib_ref[...] = gxh_ref[...] * jnp.dot(
        ex, r_ref[...], preferred_element_type=jnp.float32)


def _edge_ex_pass(p1, p2, a_e, gxh):
    return pl.pallas_call(
        _edge_ex_body,
        grid=(EP_ // BE,),
        in_specs=[pl.BlockSpec((BE, 16), lambda i: (i, 0)),
                  pl.BlockSpec((BE, 16), lambda i: (i, 0)),
                  pl.BlockSpec((BE, H), lambda i: (i, 0)),
                  pl.BlockSpec((BE, D), lambda i: (i, 0)),
                  _full((3, 16, H)), _full((3, H, 16)), _full((H, D))],
        out_specs=[pl.BlockSpec((BE, 16), lambda i: (i, 0)),
                   pl.BlockSpec((BE, D), lambda i: (i, 0))],
        out_shape=[jax.ShapeDtypeStruct((EP_, 16), jnp.float32),
                   jax.ShapeDtypeStruct((EP_, D), jnp.float32)],
    )(p1, p2, a_e, gxh, jnp.asarray(_S), jnp.asarray(_P), jnp.asarray(_R))


def _ln(x, g, b):
    mu = jnp.mean(x, -1, keepdims=True)
    xc = x - mu
    var = jnp.mean(xc * xc, -1, keepdims=True)
    return xc / jnp.sqrt(var + np.float32(1e-5)) * g + b


def _finalize_body(eu, agg_ref, den_ref, node_ref, bias_ref, g_ref, b_ref,
                   s_ref, r_ref, *rest):
    invd = 1.0 / (jnp.dot(den_ref[...], s_ref[0],
                          preferred_element_type=jnp.float32) + np.float32(1e-16))
    agg = agg_ref[...] * jnp.dot(invd, r_ref[...],
                                 preferred_element_type=jnp.float32)
    conv = _ln(agg + bias_ref[...], g_ref[...], b_ref[...])
    nn = jnp.maximum(conv, 0.0) + node_ref[...]
    if eu:
        w1a_ref, w1b_ref, b1_ref, out_ref, ns_ref, nd_ref = rest
        out_ref[...] = nn
        ns_ref[...] = jnp.dot(nn, w1a_ref[...],
                              preferred_element_type=jnp.float32) + b1_ref[...]
        nd_ref[...] = jnp.dot(nn, w1b_ref[...],
                              preferred_element_type=jnp.float32)
    else:
        rest[0][...] = nn


def _node_finalize(aggsum, denom, node, bias, g, b, w1a=None, w1b=None, b1=None):
    eu = w1a is not None
    ins = [aggsum, denom, node, bias, g, b, jnp.asarray(_S), jnp.asarray(_R)]
    in_specs = [pl.BlockSpec((BN, D), lambda i: (i, 0)),
                pl.BlockSpec((BN, 16), lambda i: (i, 0)),
                pl.BlockSpec((BN, D), lambda i: (i, 0)),
                _full((1, D)), _full((1, D)), _full((1, D)),
                _full((3, 16, H)), _full((H, D))]
    nout = 3 if eu else 1
    if eu:
        ins += [w1a, w1b, b1]
        in_specs += [_full((D, D)), _full((D, D)), _full((1, D))]
    out = pl.pallas_call(
        functools.partial(_finalize_body, eu),
        grid=(NP_ // BN,),
        in_specs=in_specs,
        out_specs=[pl.BlockSpec((BN, D), lambda i: (i, 0))] * nout,
        out_shape=[jax.ShapeDtypeStruct((NP_, D), jnp.float32)] * nout,
    )(*ins)
    return out if eu else out[0]


def _edge_up_body(gs_ref, gd_ref, e_ref, w1c_ref, w2_ref, b2_ref, g_ref,
                  b_ref, out_ref):
    i = pl.program_id(0)
    eb = e_ref[...]
    h = gs_ref[...] + gd_ref[...] + jnp.dot(
        eb, w1c_ref[...], preferred_element_type=jnp.float32)
    h = jnp.maximum(h, 0.0)
    h = jnp.dot(h, w2_ref[...], preferred_element_type=jnp.float32) + b2_ref[...]
    h = _ln(h, g_ref[...], b_ref[...])
    en = jnp.maximum(h, 0.0) + eb
    rid = i * BE + lax.broadcasted_iota(jnp.int32, (BE, 1), 0)
    out_ref[...] = jnp.where(rid < E, en, 0.0)


def _edge_update(gs, gd, edge, w1c, w2, b2, g, b):
    return pl.pallas_call(
        _edge_up_body,
        grid=(EP_ // BE,),
        in_specs=[pl.BlockSpec((BE, D), lambda i: (i, 0)),
                  pl.BlockSpec((BE, D), lambda i: (i, 0)),
                  pl.BlockSpec((BE, D), lambda i: (i, 0)),
                  _full((D, D)), _full((D, D)), _full((1, D)),
                  _full((1, D)), _full((1, D))],
        out_specs=pl.BlockSpec((BE, D), lambda i: (i, 0)),
        out_shape=jax.ShapeDtypeStruct((EP_, D), jnp.float32),
    )(gs, gd, edge, w1c, w2, b2, g, b)


def _sc_gather(table, idx2d, width, ngc=4):
    groups_per_tile = EP_ // 128 // (NC * NS)
    chunks = groups_per_tile // ngc
    mesh = plsc.VectorSubcoreMesh(core_axis_name="c", subcore_axis_name="s")

    @functools.partial(
        pl.kernel, mesh=mesh,
        out_type=jax.ShapeDtypeStruct((EP_, width), jnp.float32),
        scratch_types=[pltpu.VMEM((ngc, 128), jnp.int32),
                       pltpu.VMEM((ngc * 128, width), jnp.float32),
                       pltpu.SemaphoreType.DMA],
    )
    def k(table_h, idx_h, out_h, idx_v, rows_v, sem):
        wid = lax.axis_index("s") * NC + lax.axis_index("c")
        g0 = wid * groups_per_tile

        def body(cidx, carry):
            gb = g0 + cidx * ngc
            pltpu.sync_copy(idx_h.at[pl.ds(gb, ngc)], idx_v)
            cps = [pltpu.async_copy(table_h.at[idx_v.at[j]],
                                    rows_v.at[pl.ds(j * 128, 128)], sem)
                   for j in range(ngc)]
            for cp in cps:
                cp.wait()
            pltpu.sync_copy(rows_v, out_h.at[pl.ds(gb * 128, ngc * 128)])
            return carry

        lax.fori_loop(0, chunks, body, 0)

    return k(table, idx2d)


def _sc_scatter_add(contrib, idx2d, init, width, ngc=4):
    groups_per_tile = EP_ // 128 // NS
    chunks = groups_per_tile // ngc
    rows_per_tile = HALF // NS
    mesh = plsc.VectorSubcoreMesh(core_axis_name="c", subcore_axis_name="s")

    @functools.partial(
        pl.kernel, mesh=mesh,
        out_type=jax.ShapeDtypeStruct((NP_, width), jnp.float32),
        scratch_types=[pltpu.VMEM((ngc, 128), jnp.int32),
                       pltpu.VMEM((ngc * 128, width), jnp.float32),
                       pltpu.VMEM_SHARED((HALF + 8, width), jnp.float32),
                       pltpu.SemaphoreType.DMA],
    )
    def k(contrib_h, idx_h, init_h, out_h, idx_v, rows_v, acc_sh, sem):
        ci = lax.axis_index("c")
        si = lax.axis_index("s")
        base = ci * HALF
        r0 = si * rows_per_tile
        pltpu.sync_copy(init_h.at[pl.ds(base + r0, rows_per_tile)],
                        acc_sh.at[pl.ds(r0, rows_per_tile)])
        plsc.subcore_barrier()

        def body(cidx, carry):
            gb = si * groups_per_tile + cidx * ngc
            pltpu.sync_copy(idx_h.at[pl.ds(gb, ngc)], idx_v)
            pltpu.sync_copy(contrib_h.at[pl.ds(gb * 128, ngc * 128)], rows_v)
            for j in range(ngc):
                for t in range(8):
                    v = idx_v[j, pl.ds(t * 16, 16)] - base
                    ok = (v >= 0) & (v < HALF)
                    idx_v[j, pl.ds(t * 16, 16)] = jnp.where(
                        ok, v, jnp.full((16,), TRASH, jnp.int32))
            for j in range(ngc):
                pltpu.sync_copy(rows_v.at[pl.ds(j * 128, 128)],
                                acc_sh.at[idx_v.at[j]], add=True)
            return carry

        lax.fori_loop(0, chunks, body, 0)
        plsc.subcore_barrier()
        pltpu.sync_copy(acc_sh.at[pl.ds(r0, rows_per_tile)],
                        out_h.at[pl.ds(base + r0, rows_per_tile)])

    return k(contrib, idx2d, init)


def kernel(x, edge_index, edge_attr, atom_table, bond_table, W_lin, att_src,
           att_dst, att_edge, W_edge, gat_bias, bn_g, bn_b, ln_g, ln_b,
           eu_W1, eu_b1, eu_W2, eu_b2):
    f32 = jnp.float32
    xp = jnp.pad(x, ((0, NP_ - N), (0, 0)))
    eap = jnp.pad(edge_attr, ((0, EP_ - E), (0, 0)))
    src = edge_index[0]
    dst = edge_index[1]
    srcp = jnp.pad(src, (0, EP_ - E)).reshape(EP_ // 128, 128)
    dstg = jnp.pad(dst, (0, EP_ - E)).reshape(EP_ // 128, 128)
    dsts = jnp.pad(dst, (0, EP_ - E),
                   constant_values=np.int32(1 << 30)).reshape(EP_ // 128, 128)
    ones16 = jnp.ones((NP_, 16), f32)

    node = _encode(xp, atom_table, _AV, _AOFF, N, NP_, BN)
    edge = _encode(eap, bond_table, _BV, _BOFF, E, EP_, BE)

    for l in range(L):
        asf = att_src[l].reshape(1, D)
        adf = att_dst[l].reshape(1, D)
        aef = att_edge[l].reshape(1, D)
        we = (W_edge[l] * aef) @ jnp.asarray(_RT)
        a_e, esum8 = _edge_attn_pass(edge, we)
        xh, tbl = _node_stage(node, W_lin[l], asf, adf, we, esum8)
        p1 = _sc_gather(tbl, srcp, 16)
        p2 = _sc_gather(tbl, dstg, 16)
        gxh = _sc_gather(xh, srcp, D)
        ex16, contrib = _edge_ex_pass(p1, p2, a_e, gxh)
        denom = _sc_scatter_add(ex16, dsts, ones16, 16)
        aggsum = _sc_scatter_add(contrib, dsts, xh, D)
        bias = gat_bias[l].reshape(1, D)
        g = bn_g[l].reshape(1, D)
        b = bn_b[l].reshape(1, D)
        if l < L - 1:
            node, ns_tbl, nd_tbl = _node_finalize(
                aggsum, denom, node, bias, g, b,
                eu_W1[l][0:D], eu_W1[l][D:2 * D], eu_b1[l].reshape(1, D))
            gs = _sc_gather(ns_tbl, srcp, D)
            gd = _sc_gather(nd_tbl, dstg, D)
            edge = _edge_update(gs, gd, edge, eu_W1[l][2 * D:3 * D],
                                eu_W2[l], eu_b2[l].reshape(1, D),
                                ln_g[l].reshape(1, D), ln_b[l].reshape(1, D))
        else:
            node = _node_finalize(aggsum, denom, node, bias, g, b)
    return node[:N]


# speedup vs baseline: 14.8335x; 14.8335x over previous
import functools

import jax
import jax.numpy as jnp
import numpy as np
from jax import lax
from jax.experimental import pallas as pl
from jax.experimental.pallas import tpu as pltpu
from jax.experimental.pallas import tpu_sc as plsc

N = 50000
E = 800000
D = 64
H = 4
C = 16
L = 4
NEG = 0.2
_ATOM_DIMS = [119, 4, 12, 12, 10, 6, 6, 2, 2]
_BOND_DIMS = [5, 6, 2]
_AOFF = np.concatenate([[0], np.cumsum(_ATOM_DIMS)[:-1]]).astype(np.int32)
_BOFF = np.concatenate([[0], np.cumsum(_BOND_DIMS)[:-1]]).astype(np.int32)
_AV = int(sum(_ATOM_DIMS))
_BV = int(sum(_BOND_DIMS))

NP_ = 50048
EP_ = 819200
BN = 6256
BE = 8192
NC, NS = 2, 16
HALF = NP_ // NC
TRASH = HALF

_R = np.zeros((H, D), np.float32)
for h in range(H):
    _R[h, h * C:(h + 1) * C] = 1.0
_RT = _R.T.copy()
_P = np.zeros((3, H, 16), np.float32)
for j in range(3):
    for h in range(H):
        _P[j, h, j * H + h] = 1.0
_S = np.zeros((3, 16, H), np.float32)
for j in range(3):
    for h in range(H):
        _S[j, j * H + h, h] = 1.0


def _full(shape):
    nd = len(shape)
    return pl.BlockSpec(shape, lambda i, _n=nd: (0,) * _n)


def _enc_body(nv, offs, nvalid, blk, x_ref, tab_ref, out_ref):
    i = pl.program_id(0)
    xb = x_ref[...]
    oh = jnp.zeros((blk, nv), jnp.float32)
    iot = lax.broadcasted_iota(jnp.int32, (blk, nv), 1)
    for k in range(len(offs)):
        col = lax.slice(xb, (0, k), (blk, k + 1)) + np.int32(offs[k])
        oh = oh + (col == iot).astype(jnp.float32)
    node = jnp.dot(oh, tab_ref[...], preferred_element_type=jnp.float32)
    rid = i * blk + lax.broadcasted_iota(jnp.int32, (blk, 1), 0)
    out_ref[...] = jnp.where(rid < nvalid, node, 0.0)


def _encode(x_pad, table, nvals, offs, nvalid, rows, blk):
    body = functools.partial(_enc_body, nvals, offs, nvalid, blk)
    return pl.pallas_call(
        body,
        grid=(rows // blk,),
        in_specs=[pl.BlockSpec((blk, x_pad.shape[1]), lambda i: (i, 0)),
                  _full(table.shape)],
        out_specs=pl.BlockSpec((blk, D), lambda i: (i, 0)),
        out_shape=jax.ShapeDtypeStruct((rows, D), jnp.float32),
    )(x_pad, table)


def _attn_pass_body(e_ref, we_ref, ae_ref, esum_ref):
    i = pl.program_id(0)
    eb = e_ref[...]
    ae_ref[...] = jnp.dot(eb, we_ref[...], preferred_element_type=jnp.float32)
    ones8 = jnp.ones((8, BE), jnp.float32)
    part = jnp.dot(ones8, eb, preferred_element_type=jnp.float32)

    @pl.when(i == 0)
    def _():
        esum_ref[...] = jnp.zeros_like(esum_ref)

    esum_ref[...] += part


def _edge_attn_pass(edge, we):
    return pl.pallas_call(
        _attn_pass_body,
        grid=(EP_ // BE,),
        in_specs=[pl.BlockSpec((BE, D), lambda i: (i, 0)), _full((D, H))],
        out_specs=[pl.BlockSpec((BE, H), lambda i: (i, 0)),
                   pl.BlockSpec((8, D), lambda i: (0, 0))],
        out_shape=[jax.ShapeDtypeStruct((EP_, H), jnp.float32),
                   jax.ShapeDtypeStruct((8, D), jnp.float32)],
    )(edge, we)


def _node_stage_body(node_ref, wl_ref, asf_ref, adf_ref, we_ref, esum_ref,
                     rt_ref, p_ref, xh_ref, tbl_ref):
    xh = jnp.dot(node_ref[...], wl_ref[...], preferred_element_type=jnp.float32)
    rt = rt_ref[...]
    a_s = jnp.dot(xh * asf_ref[...], rt, preferred_element_type=jnp.float32)
    a_d = jnp.dot(xh * adf_ref[...], rt, preferred_element_type=jnp.float32)
    e_mean = jnp.sum(esum_ref[...], 0, keepdims=True) * np.float32(1.0 / (8 * E))
    a_el = jnp.dot(e_mean, we_ref[...], preferred_element_type=jnp.float32)
    pre = a_s + a_d + a_el
    sub = jnp.where(pre >= 0, pre, pre * np.float32(NEG))
    tbl = (jnp.dot(a_s, p_ref[0], preferred_element_type=jnp.float32)
           + jnp.dot(a_d, p_ref[1], preferred_element_type=jnp.float32)
           + jnp.dot(sub, p_ref[2], preferred_element_type=jnp.float32))
    xh_ref[...] = xh
    tbl_ref[...] = tbl


def _node_stage(node, w_lin, asf, adf, we, esum8):
    return pl.pallas_call(
        _node_stage_body,
        grid=(NP_ // BN,),
        in_specs=[pl.BlockSpec((BN, D), lambda i: (i, 0)), _full((D, D)),
                  _full((1, D)), _full((1, D)), _full((D, H)), _full((8, D)),
                  _full((D, H)), _full((3, H, 16))],
        out_specs=[pl.BlockSpec((BN, D), lambda i: (i, 0)),
                   pl.BlockSpec((BN, 16), lambda i: (i, 0))],
        out_shape=[jax.ShapeDtypeStruct((NP_, D), jnp.float32),
                   jax.ShapeDtypeStruct((NP_, 16), jnp.float32)],
    )(node, w_lin, asf, adf, we, esum8, jnp.asarray(_RT), jnp.asarray(_P))


def _edge_ex_body(p1_ref, p2_ref, ae_ref, gxh_ref, s_ref, p_ref, r_ref,
                  ex16_ref, contrib_ref):
    p1 = p1_ref[...]
    p2 = p2_ref[...]
    a = (jnp.dot(p1, s_ref[0], preferred_element_type=jnp.float32)
         + jnp.dot(p2, s_ref[1], preferred_element_type=jnp.float32)
         + ae_ref[...])
    a = jnp.where(a >= 0, a, a * np.float32(NEG))
    sub = jnp.dot(p2, s_ref[2], preferred_element_type=jnp.float32)
    ex = jnp.exp(a - sub)
    ex16_ref[...] = jnp.dot(ex, p_ref[0], preferred_element_type=jnp.float32)
    contrib_ref[...] = gxh_ref[...] * jnp.dot(
        ex, r_ref[...], preferred_element_type=jnp.float32)


def _edge_ex_pass(p1, p2, a_e, gxh):
    return pl.pallas_call(
        _edge_ex_body,
        grid=(EP_ // BE,),
        in_specs=[pl.BlockSpec((BE, 16), lambda i: (i, 0)),
                  pl.BlockSpec((BE, 16), lambda i: (i, 0)),
                  pl.BlockSpec((BE, H), lambda i: (i, 0)),
                  pl.BlockSpec((BE, D), lambda i: (i, 0)),
                  _full((3, 16, H)), _full((3, H, 16)), _full((H, D))],
        out_specs=[pl.BlockSpec((BE, 16), lambda i: (i, 0)),
                   pl.BlockSpec((BE, D), lambda i: (i, 0))],
        out_shape=[jax.ShapeDtypeStruct((EP_, 16), jnp.float32),
                   jax.ShapeDtypeStruct((EP_, D), jnp.float32)],
    )(p1, p2, a_e, gxh, jnp.asarray(_S), jnp.asarray(_P), jnp.asarray(_R))


def _ln(x, g, b):
    mu = jnp.mean(x, -1, keepdims=True)
    xc = x - mu
    var = jnp.mean(xc * xc, -1, keepdims=True)
    return xc / jnp.sqrt(var + np.float32(1e-5)) * g + b


def _finalize_body(eu, agg_ref, den_ref, node_ref, bias_ref, g_ref, b_ref,
                   s_ref, r_ref, *rest):
    invd = 1.0 / (jnp.dot(den_ref[...], s_ref[0],
                          preferred_element_type=jnp.float32) + np.float32(1e-16))
    agg = agg_ref[...] * jnp.dot(invd, r_ref[...],
                                 preferred_element_type=jnp.float32)
    conv = _ln(agg + bias_ref[...], g_ref[...], b_ref[...])
    nn = jnp.maximum(conv, 0.0) + node_ref[...]
    if eu:
        w1a_ref, w1b_ref, b1_ref, out_ref, ns_ref, nd_ref = rest
        out_ref[...] = nn
        ns_ref[...] = jnp.dot(nn, w1a_ref[...],
                              preferred_element_type=jnp.float32) + b1_ref[...]
        nd_ref[...] = jnp.dot(nn, w1b_ref[...],
                              preferred_element_type=jnp.float32)
    else:
        rest[0][...] = nn


def _node_finalize(aggsum, denom, node, bias, g, b, w1a=None, w1b=None, b1=None):
    eu = w1a is not None
    ins = [aggsum, denom, node, bias, g, b, jnp.asarray(_S), jnp.asarray(_R)]
    in_specs = [pl.BlockSpec((BN, D), lambda i: (i, 0)),
                pl.BlockSpec((BN, 16), lambda i: (i, 0)),
                pl.BlockSpec((BN, D), lambda i: (i, 0)),
                _full((1, D)), _full((1, D)), _full((1, D)),
                _full((3, 16, H)), _full((H, D))]
    nout = 3 if eu else 1
    if eu:
        ins += [w1a, w1b, b1]
        in_specs += [_full((D, D)), _full((D, D)), _full((1, D))]
    out = pl.pallas_call(
        functools.partial(_finalize_body, eu),
        grid=(NP_ // BN,),
        in_specs=in_specs,
        out_specs=[pl.BlockSpec((BN, D), lambda i: (i, 0))] * nout,
        out_shape=[jax.ShapeDtypeStruct((NP_, D), jnp.float32)] * nout,
    )(*ins)
    return out if eu else out[0]


def _edge_up_body(gs_ref, gd_ref, e_ref, w1c_ref, w2_ref, b2_ref, g_ref,
                  b_ref, out_ref):
    i = pl.program_id(0)
    eb = e_ref[...]
    h = gs_ref[...] + gd_ref[...] + jnp.dot(
        eb, w1c_ref[...], preferred_element_type=jnp.float32)
    h = jnp.maximum(h, 0.0)
    h = jnp.dot(h, w2_ref[...], preferred_element_type=jnp.float32) + b2_ref[...]
    h = _ln(h, g_ref[...], b_ref[...])
    en = jnp.maximum(h, 0.0) + eb
    rid = i * BE + lax.broadcasted_iota(jnp.int32, (BE, 1), 0)
    out_ref[...] = jnp.where(rid < E, en, 0.0)


def _edge_update(gs, gd, edge, w1c, w2, b2, g, b):
    return pl.pallas_call(
        _edge_up_body,
        grid=(EP_ // BE,),
        in_specs=[pl.BlockSpec((BE, D), lambda i: (i, 0)),
                  pl.BlockSpec((BE, D), lambda i: (i, 0)),
                  pl.BlockSpec((BE, D), lambda i: (i, 0)),
                  _full((D, D)), _full((D, D)), _full((1, D)),
                  _full((1, D)), _full((1, D))],
        out_specs=pl.BlockSpec((BE, D), lambda i: (i, 0)),
        out_shape=jax.ShapeDtypeStruct((EP_, D), jnp.float32),
    )(gs, gd, edge, w1c, w2, b2, g, b)


def _sc_gather(table, idx2d, width, ngc=4):
    groups_per_tile = EP_ // 128 // (NC * NS)
    chunks = groups_per_tile // ngc
    mesh = plsc.VectorSubcoreMesh(core_axis_name="c", subcore_axis_name="s")

    @functools.partial(
        pl.kernel, mesh=mesh,
        out_type=jax.ShapeDtypeStruct((EP_, width), jnp.float32),
        compiler_params=pltpu.CompilerParams(use_tc_tiling_on_sc=False),
        scratch_types=[pltpu.VMEM((ngc, 128), jnp.int32),
                       pltpu.VMEM((ngc * 128, width), jnp.float32),
                       pltpu.SemaphoreType.DMA],
    )
    def k(table_h, idx_h, out_h, idx_v, rows_v, sem):
        wid = lax.axis_index("s") * NC + lax.axis_index("c")
        g0 = wid * groups_per_tile

        def body(cidx, carry):
            gb = g0 + cidx * ngc
            pltpu.sync_copy(idx_h.at[pl.ds(gb, ngc)], idx_v)
            cps = [pltpu.async_copy(table_h.at[idx_v.at[j]],
                                    rows_v.at[pl.ds(j * 128, 128)], sem)
                   for j in range(ngc)]
            for cp in cps:
                cp.wait()
            pltpu.sync_copy(rows_v, out_h.at[pl.ds(gb * 128, ngc * 128)])
            return carry

        lax.fori_loop(0, chunks, body, 0)

    return k(table, idx2d)


def _sc_scatter_add(contrib, idx2d, init, width, ngc=4):
    groups_per_tile = EP_ // 128 // NS
    chunks = groups_per_tile // ngc
    nseg = 1 if width <= 24 else 2
    segsz = HALF // nseg
    trash = segsz
    rows_per_tile = segsz // NS
    mesh = plsc.VectorSubcoreMesh(core_axis_name="c", subcore_axis_name="s")

    @functools.partial(
        pl.kernel, mesh=mesh,
        out_type=jax.ShapeDtypeStruct((NP_, width), jnp.float32),
        compiler_params=pltpu.CompilerParams(use_tc_tiling_on_sc=False),
        scratch_types=[pltpu.VMEM((ngc, 128), jnp.int32),
                       pltpu.VMEM((ngc * 128, width), jnp.float32),
                       pltpu.VMEM_SHARED((segsz + 8, width), jnp.float32),
                       pltpu.SemaphoreType.DMA],
    )
    def k(contrib_h, idx_h, init_h, out_h, idx_v, rows_v, acc_sh, sem):
        ci = lax.axis_index("c")
        si = lax.axis_index("s")
        r0 = si * rows_per_tile
        for p in range(nseg):
            base = ci * HALF + p * segsz
            pltpu.sync_copy(init_h.at[pl.ds(base + r0, rows_per_tile)],
                            acc_sh.at[pl.ds(r0, rows_per_tile)])
            plsc.subcore_barrier()

            def body(cidx, carry):
                gb = si * groups_per_tile + cidx * ngc
                pltpu.sync_copy(idx_h.at[pl.ds(gb, ngc)], idx_v)
                pltpu.sync_copy(contrib_h.at[pl.ds(gb * 128, ngc * 128)],
                                rows_v)
                for j in range(ngc):
                    for t in range(8):
                        v = idx_v[j, pl.ds(t * 16, 16)] - base
                        ok = (v >= 0) & (v < segsz)
                        idx_v[j, pl.ds(t * 16, 16)] = jnp.where(
                            ok, v, jnp.full((16,), trash, jnp.int32))
                for j in range(ngc):
                    pltpu.sync_copy(rows_v.at[pl.ds(j * 128, 128)],
                                    acc_sh.at[idx_v.at[j]], add=True)
                return carry

            lax.fori_loop(0, chunks, body, 0)
            plsc.subcore_barrier()
            pltpu.sync_copy(acc_sh.at[pl.ds(r0, rows_per_tile)],
                            out_h.at[pl.ds(base + r0, rows_per_tile)])
            plsc.subcore_barrier()

    return k(contrib, idx2d, init)


def kernel(x, edge_index, edge_attr, atom_table, bond_table, W_lin, att_src,
           att_dst, att_edge, W_edge, gat_bias, bn_g, bn_b, ln_g, ln_b,
           eu_W1, eu_b1, eu_W2, eu_b2):
    f32 = jnp.float32
    xp = jnp.pad(x, ((0, NP_ - N), (0, 0)))
    eap = jnp.pad(edge_attr, ((0, EP_ - E), (0, 0)))
    src = edge_index[0]
    dst = edge_index[1]
    srcp = jnp.pad(src, (0, EP_ - E)).reshape(EP_ // 128, 128)
    dstg = jnp.pad(dst, (0, EP_ - E)).reshape(EP_ // 128, 128)
    dsts = jnp.pad(dst, (0, EP_ - E),
                   constant_values=np.int32(1 << 30)).reshape(EP_ // 128, 128)
    ones16 = jnp.ones((NP_, 16), f32)

    node = _encode(xp, atom_table, _AV, _AOFF, N, NP_, BN)
    edge = _encode(eap, bond_table, _BV, _BOFF, E, EP_, BE)

    for l in range(L):
        asf = att_src[l].reshape(1, D)
        adf = att_dst[l].reshape(1, D)
        aef = att_edge[l].reshape(1, D)
        we = (W_edge[l] * aef) @ jnp.asarray(_RT)
        a_e, esum8 = _edge_attn_pass(edge, we)
        xh, tbl = _node_stage(node, W_lin[l], asf, adf, we, esum8)
        p1 = _sc_gather(tbl, srcp, 16)
        p2 = _sc_gather(tbl, dstg, 16)
        gxh = _sc_gather(xh, srcp, D)
        ex16, contrib = _edge_ex_pass(p1, p2, a_e, gxh)
        denom = _sc_scatter_add(ex16, dsts, ones16, 16)
        aggsum = _sc_scatter_add(contrib, dsts, xh, D)
        bias = gat_bias[l].reshape(1, D)
        g = bn_g[l].reshape(1, D)
        b = bn_b[l].reshape(1, D)
        if l < L - 1:
            node, ns_tbl, nd_tbl = _node_finalize(
                aggsum, denom, node, bias, g, b,
                eu_W1[l][0:D], eu_W1[l][D:2 * D], eu_b1[l].reshape(1, D))
            gs = _sc_gather(ns_tbl, srcp, D)
            gd = _sc_gather(nd_tbl, dstg, D)
            edge = _edge_update(gs, gd, edge, eu_W1[l][2 * D:3 * D],
                                eu_W2[l], eu_b2[l].reshape(1, D),
                                ln_g[l].reshape(1, D), ln_b[l].reshape(1, D))
        else:
            node = _node_finalize(aggsum, denom, node, bias, g, b)
    return node[:N]


# speedup vs baseline: 15.5232x; 1.0465x over previous
import functools

import jax
import jax.numpy as jnp
import numpy as np
from jax import lax
from jax.experimental import pallas as pl
from jax.experimental.pallas import tpu as pltpu
from jax.experimental.pallas import tpu_sc as plsc

N = 50000
E = 800000
D = 64
H = 4
C = 16
L = 4
NEG = 0.2
_ATOM_DIMS = [119, 4, 12, 12, 10, 6, 6, 2, 2]
_BOND_DIMS = [5, 6, 2]
_AOFF = np.concatenate([[0], np.cumsum(_ATOM_DIMS)[:-1]]).astype(np.int32)
_BOFF = np.concatenate([[0], np.cumsum(_BOND_DIMS)[:-1]]).astype(np.int32)
_AV = int(sum(_ATOM_DIMS))
_BV = int(sum(_BOND_DIMS))

NP_ = 50048
EP_ = 819200
BN = 3128
BE = 4096
NC, NS = 2, 16
HALF = NP_ // NC
TRASH = HALF

_R = np.zeros((H, D), np.float32)
for h in range(H):
    _R[h, h * C:(h + 1) * C] = 1.0
_RT = _R.T.copy()
_P = np.zeros((3, H, 16), np.float32)
for j in range(3):
    for h in range(H):
        _P[j, h, j * H + h] = 1.0
_S = np.zeros((3, 16, H), np.float32)
for j in range(3):
    for h in range(H):
        _S[j, j * H + h, h] = 1.0
_E64 = np.eye(64, dtype=np.float32)
_SA = _E64[:, 0:24].copy()
_SB = _E64[:, 24:48].copy()
_SC = _E64[:, 48:64].copy()
_RA = _R[:, 0:24].copy()
_RB = _R[:, 24:48].copy()
_RC = _R[:, 48:64].copy()
_E16TO24 = np.zeros((16, 24), np.float32)
_E16TO24[:16, :16] = np.eye(16)
_P4TO24 = np.zeros((4, 24), np.float32)
for h in range(H):
    _P4TO24[h, 16 + h] = 1.0
_SCI = np.zeros((64, 24), np.float32)
_SCI[48:, :16] = np.eye(16)
_ONESC = np.zeros((1, 24), np.float32)
_ONESC[0, 16:20] = 1.0
_TA = _SA.T.copy()
_TB = _SB.T.copy()
_TCM = np.zeros((24, 64), np.float32)
_TCM[:16, 48:] = np.eye(16)
_TD = np.zeros((24, 4), np.float32)
for h in range(H):
    _TD[16 + h, h] = 1.0


def _full(shape):
    nd = len(shape)
    return pl.BlockSpec(shape, lambda i, _n=nd: (0,) * _n)


def _enc_body(nv, offs, nvalid, blk, x_ref, tab_ref, out_ref):
    i = pl.program_id(0)
    xb = x_ref[...]
    oh = jnp.zeros((blk, nv), jnp.float32)
    iot = lax.broadcasted_iota(jnp.int32, (blk, nv), 1)
    for k in range(len(offs)):
        col = lax.slice(xb, (0, k), (blk, k + 1)) + np.int32(offs[k])
        oh = oh + (col == iot).astype(jnp.float32)
    node = jnp.dot(oh, tab_ref[...], preferred_element_type=jnp.float32)
    rid = i * blk + lax.broadcasted_iota(jnp.int32, (blk, 1), 0)
    out_ref[...] = jnp.where(rid < nvalid, node, 0.0)


def _encode(x_pad, table, nvals, offs, nvalid, rows, blk):
    body = functools.partial(_enc_body, nvals, offs, nvalid, blk)
    return pl.pallas_call(
        body,
        grid=(rows // blk,),
        in_specs=[pl.BlockSpec((blk, x_pad.shape[1]), lambda i: (i, 0)),
                  _full(table.shape)],
        out_specs=pl.BlockSpec((blk, D), lambda i: (i, 0)),
        out_shape=jax.ShapeDtypeStruct((rows, D), jnp.float32),
    )(x_pad, table)


def _attn_pass_body(e_ref, we_ref, ae_ref, esum_ref):
    i = pl.program_id(0)
    eb = e_ref[...]
    ae_ref[...] = jnp.dot(eb, we_ref[...], preferred_element_type=jnp.float32)
    ones8 = jnp.ones((8, BE), jnp.float32)
    part = jnp.dot(ones8, eb, preferred_element_type=jnp.float32)

    @pl.when(i == 0)
    def _():
        esum_ref[...] = jnp.zeros_like(esum_ref)

    esum_ref[...] += part


def _edge_attn_pass(edge, we):
    return pl.pallas_call(
        _attn_pass_body,
        grid=(EP_ // BE,),
        in_specs=[pl.BlockSpec((BE, D), lambda i: (i, 0)), _full((D, H))],
        out_specs=[pl.BlockSpec((BE, H), lambda i: (i, 0)),
                   pl.BlockSpec((8, D), lambda i: (0, 0))],
        out_shape=[jax.ShapeDtypeStruct((EP_, H), jnp.float32),
                   jax.ShapeDtypeStruct((8, D), jnp.float32)],
    )(edge, we)


def _node_stage_body(node_ref, wl_ref, asf_ref, adf_ref, we_ref, esum_ref,
                     rt_ref, p_ref, sa_ref, sb_ref, sci_ref, onesc_ref,
                     xh_ref, tbl_ref, xha_ref, xhb_ref, xhc_ref):
    xh = jnp.dot(node_ref[...], wl_ref[...], preferred_element_type=jnp.float32)
    rt = rt_ref[...]
    a_s = jnp.dot(xh * asf_ref[...], rt, preferred_element_type=jnp.float32)
    a_d = jnp.dot(xh * adf_ref[...], rt, preferred_element_type=jnp.float32)
    e_mean = jnp.sum(esum_ref[...], 0, keepdims=True) * np.float32(1.0 / (8 * E))
    a_el = jnp.dot(e_mean, we_ref[...], preferred_element_type=jnp.float32)
    pre = a_s + a_d + a_el
    sub = jnp.where(pre >= 0, pre, pre * np.float32(NEG))
    tbl = (jnp.dot(a_s, p_ref[0], preferred_element_type=jnp.float32)
           + jnp.dot(a_d, p_ref[1], preferred_element_type=jnp.float32)
           + jnp.dot(sub, p_ref[2], preferred_element_type=jnp.float32))
    xh_ref[...] = xh
    tbl_ref[...] = tbl
    xha_ref[...] = jnp.dot(xh, sa_ref[...], preferred_element_type=jnp.float32)
    xhb_ref[...] = jnp.dot(xh, sb_ref[...], preferred_element_type=jnp.float32)
    xhc_ref[...] = jnp.dot(xh, sci_ref[...],
                           preferred_element_type=jnp.float32) + onesc_ref[...]


def _node_stage(node, w_lin, asf, adf, we, esum8):
    return pl.pallas_call(
        _node_stage_body,
        grid=(NP_ // BN,),
        in_specs=[pl.BlockSpec((BN, D), lambda i: (i, 0)), _full((D, D)),
                  _full((1, D)), _full((1, D)), _full((D, H)), _full((8, D)),
                  _full((D, H)), _full((3, H, 16)),
                  _full((D, 24)), _full((D, 24)), _full((D, 24)),
                  _full((1, 24))],
        out_specs=[pl.BlockSpec((BN, D), lambda i: (i, 0)),
                   pl.BlockSpec((BN, 16), lambda i: (i, 0)),
                   pl.BlockSpec((BN, 24), lambda i: (i, 0)),
                   pl.BlockSpec((BN, 24), lambda i: (i, 0)),
                   pl.BlockSpec((BN, 24), lambda i: (i, 0))],
        out_shape=[jax.ShapeDtypeStruct((NP_, D), jnp.float32),
                   jax.ShapeDtypeStruct((NP_, 16), jnp.float32),
                   jax.ShapeDtypeStruct((NP_, 24), jnp.float32),
                   jax.ShapeDtypeStruct((NP_, 24), jnp.float32),
                   jax.ShapeDtypeStruct((NP_, 24), jnp.float32)],
    )(node, w_lin, asf, adf, we, esum8, jnp.asarray(_RT), jnp.asarray(_P),
      jnp.asarray(_SA), jnp.asarray(_SB), jnp.asarray(_SCI),
      jnp.asarray(_ONESC))


def _edge_ex_body(p1_ref, p2_ref, ae_ref, gxh_ref, s_ref, ra_ref,
                  rb_ref, rc_ref, sa_ref, sb_ref, sc_ref, e24_ref, p24_ref,
                  ca_ref, cb_ref, cc_ref):
    p1 = p1_ref[...]
    p2 = p2_ref[...]
    a = (jnp.dot(p1, s_ref[0], preferred_element_type=jnp.float32)
         + jnp.dot(p2, s_ref[1], preferred_element_type=jnp.float32)
         + ae_ref[...])
    a = jnp.where(a >= 0, a, a * np.float32(NEG))
    sub = jnp.dot(p2, s_ref[2], preferred_element_type=jnp.float32)
    ex = jnp.exp(a - sub)
    gxh = gxh_ref[...]
    ca_ref[...] = jnp.dot(gxh, sa_ref[...],
                          preferred_element_type=jnp.float32) * jnp.dot(
        ex, ra_ref[...], preferred_element_type=jnp.float32)
    cb_ref[...] = jnp.dot(gxh, sb_ref[...],
                          preferred_element_type=jnp.float32) * jnp.dot(
        ex, rb_ref[...], preferred_element_type=jnp.float32)
    cpart = jnp.dot(gxh, sc_ref[...],
                    preferred_element_type=jnp.float32) * jnp.dot(
        ex, rc_ref[...], preferred_element_type=jnp.float32)
    cc_ref[...] = (jnp.dot(cpart, e24_ref[...],
                           preferred_element_type=jnp.float32)
                   + jnp.dot(ex, p24_ref[...],
                             preferred_element_type=jnp.float32))


def _edge_ex_pass(p1, p2, a_e, gxh):
    return pl.pallas_call(
        _edge_ex_body,
        grid=(EP_ // BE,),
        in_specs=[pl.BlockSpec((BE, 16), lambda i: (i, 0)),
                  pl.BlockSpec((BE, 16), lambda i: (i, 0)),
                  pl.BlockSpec((BE, H), lambda i: (i, 0)),
                  pl.BlockSpec((BE, D), lambda i: (i, 0)),
                  _full((3, 16, H)), _full((H, 24)), _full((H, 24)),
                  _full((H, 16)), _full((D, 24)), _full((D, 24)),
                  _full((D, 16)), _full((16, 24)), _full((H, 24))],
        out_specs=[pl.BlockSpec((BE, 24), lambda i: (i, 0)),
                   pl.BlockSpec((BE, 24), lambda i: (i, 0)),
                   pl.BlockSpec((BE, 24), lambda i: (i, 0))],
        out_shape=[jax.ShapeDtypeStruct((EP_, 24), jnp.float32),
                   jax.ShapeDtypeStruct((EP_, 24), jnp.float32),
                   jax.ShapeDtypeStruct((EP_, 24), jnp.float32)],
    )(p1, p2, a_e, gxh, jnp.asarray(_S), jnp.asarray(_RA),
      jnp.asarray(_RB), jnp.asarray(_RC), jnp.asarray(_SA),
      jnp.asarray(_SB), jnp.asarray(_SC), jnp.asarray(_E16TO24),
      jnp.asarray(_P4TO24))


def _ln(x, g, b):
    mu = jnp.mean(x, -1, keepdims=True)
    xc = x - mu
    var = jnp.mean(xc * xc, -1, keepdims=True)
    return xc / jnp.sqrt(var + np.float32(1e-5)) * g + b


def _finalize_body(eu, agga_ref, aggb_ref, aggc_ref, node_ref, bias_ref,
                   g_ref, b_ref, ta_ref, tb_ref, tcm_ref, td_ref, r_ref,
                   *rest):
    aggc = aggc_ref[...]
    invd = 1.0 / (jnp.dot(aggc, td_ref[...],
                          preferred_element_type=jnp.float32) + np.float32(1e-16))
    aggsum = (jnp.dot(agga_ref[...], ta_ref[...],
                      preferred_element_type=jnp.float32)
              + jnp.dot(aggb_ref[...], tb_ref[...],
                        preferred_element_type=jnp.float32)
              + jnp.dot(aggc, tcm_ref[...],
                        preferred_element_type=jnp.float32))
    agg = aggsum * jnp.dot(invd, r_ref[...],
                           preferred_element_type=jnp.float32)
    conv = _ln(agg + bias_ref[...], g_ref[...], b_ref[...])
    nn = jnp.maximum(conv, 0.0) + node_ref[...]
    if eu:
        w1a_ref, w1b_ref, b1_ref, out_ref, ns_ref, nd_ref = rest
        out_ref[...] = nn
        ns_ref[...] = jnp.dot(nn, w1a_ref[...],
                              preferred_element_type=jnp.float32) + b1_ref[...]
        nd_ref[...] = jnp.dot(nn, w1b_ref[...],
                              preferred_element_type=jnp.float32)
    else:
        rest[0][...] = nn


def _node_finalize(agga, aggb, aggc, node, bias, g, b,
                   w1a=None, w1b=None, b1=None):
    eu = w1a is not None
    ins = [agga, aggb, aggc, node, bias, g, b, jnp.asarray(_TA),
           jnp.asarray(_TB), jnp.asarray(_TCM), jnp.asarray(_TD),
           jnp.asarray(_R)]
    in_specs = [pl.BlockSpec((BN, 24), lambda i: (i, 0)),
                pl.BlockSpec((BN, 24), lambda i: (i, 0)),
                pl.BlockSpec((BN, 24), lambda i: (i, 0)),
                pl.BlockSpec((BN, D), lambda i: (i, 0)),
                _full((1, D)), _full((1, D)), _full((1, D)),
                _full((24, D)), _full((24, D)), _full((24, D)),
                _full((24, H)), _full((H, D))]
    nout = 3 if eu else 1
    if eu:
        ins += [w1a, w1b, b1]
        in_specs += [_full((D, D)), _full((D, D)), _full((1, D))]
    out = pl.pallas_call(
        functools.partial(_finalize_body, eu),
        grid=(NP_ // BN,),
        in_specs=in_specs,
        out_specs=[pl.BlockSpec((BN, D), lambda i: (i, 0))] * nout,
        out_shape=[jax.ShapeDtypeStruct((NP_, D), jnp.float32)] * nout,
    )(*ins)
    return out if eu else out[0]


def _edge_up_body(gs_ref, gd_ref, e_ref, w1c_ref, w2_ref, b2_ref, g_ref,
                  b_ref, out_ref):
    i = pl.program_id(0)
    eb = e_ref[...]
    h = gs_ref[...] + gd_ref[...] + jnp.dot(
        eb, w1c_ref[...], preferred_element_type=jnp.float32)
    h = jnp.maximum(h, 0.0)
    h = jnp.dot(h, w2_ref[...], preferred_element_type=jnp.float32) + b2_ref[...]
    h = _ln(h, g_ref[...], b_ref[...])
    en = jnp.maximum(h, 0.0) + eb
    rid = i * BE + lax.broadcasted_iota(jnp.int32, (BE, 1), 0)
    out_ref[...] = jnp.where(rid < E, en, 0.0)


def _edge_update(gs, gd, edge, w1c, w2, b2, g, b):
    return pl.pallas_call(
        _edge_up_body,
        grid=(EP_ // BE,),
        in_specs=[pl.BlockSpec((BE, D), lambda i: (i, 0)),
                  pl.BlockSpec((BE, D), lambda i: (i, 0)),
                  pl.BlockSpec((BE, D), lambda i: (i, 0)),
                  _full((D, D)), _full((D, D)), _full((1, D)),
                  _full((1, D)), _full((1, D))],
        out_specs=pl.BlockSpec((BE, D), lambda i: (i, 0)),
        out_shape=jax.ShapeDtypeStruct((EP_, D), jnp.float32),
    )(gs, gd, edge, w1c, w2, b2, g, b)


def _sc_gather(table, idx2d, width):
    ngc = 5 if width >= 64 else 10
    groups_per_tile = EP_ // 128 // (NC * NS)
    chunks = groups_per_tile // ngc
    mesh = plsc.VectorSubcoreMesh(core_axis_name="c", subcore_axis_name="s")

    @functools.partial(
        pl.kernel, mesh=mesh,
        out_type=jax.ShapeDtypeStruct((EP_, width), jnp.float32),
        compiler_params=pltpu.CompilerParams(use_tc_tiling_on_sc=False),
        scratch_types=[pltpu.VMEM((2, ngc, 128), jnp.int32),
                       pltpu.VMEM((2, ngc * 128, width), jnp.float32),
                       pltpu.SemaphoreType.DMA,
                       pltpu.SemaphoreType.DMA,
                       pltpu.SemaphoreType.DMA],
    )
    def k(table_h, idx_h, out_h, idx_v, rows_v, isem, gsem, osem):
        wid = lax.axis_index("s") * NC + lax.axis_index("c")
        g0 = wid * groups_per_tile

        def idx_cp(c, b):
            return pltpu.make_async_copy(
                idx_h.at[pl.ds(g0 + c * ngc, ngc)], idx_v.at[b], isem)

        def out_cp(c, b):
            return pltpu.make_async_copy(
                rows_v.at[b],
                out_h.at[pl.ds((g0 + c * ngc) * 128, ngc * 128)], osem)

        idx_cp(0, 0).start()

        def body(c2, carry):
            for b in range(2):
                c = 2 * c2 + b
                idx_cp(c, b).wait()

                @pl.when(c + 1 < chunks)
                def _():
                    idx_cp(c + 1, 1 - b).start()

                @pl.when(c >= 2)
                def _():
                    out_cp(c - 2, b).wait()

                cps = [pltpu.async_copy(table_h.at[idx_v.at[b, j]],
                                        rows_v.at[b, pl.ds(j * 128, 128)],
                                        gsem)
                       for j in range(ngc)]
                for cp in cps:
                    cp.wait()
                out_cp(c, b).start()
            return carry

        lax.fori_loop(0, chunks // 2, body, 0)
        out_cp(chunks - 2, 0).wait()
        out_cp(chunks - 1, 1).wait()

    return k(table, idx2d)


def _sc_scatter_add(contrib, idx2d, init, width):
    ngc = 10
    groups_per_tile = EP_ // 128 // NS
    chunks = groups_per_tile // ngc
    trash = HALF
    rows_per_tile = HALF // NS
    mesh = plsc.VectorSubcoreMesh(core_axis_name="c", subcore_axis_name="s")

    @functools.partial(
        pl.kernel, mesh=mesh,
        out_type=jax.ShapeDtypeStruct((NP_, width), jnp.float32),
        compiler_params=pltpu.CompilerParams(use_tc_tiling_on_sc=False),
        scratch_types=[pltpu.VMEM((2, ngc, 128), jnp.int32),
                       pltpu.VMEM((2, ngc * 128, width), jnp.float32),
                       pltpu.VMEM_SHARED((HALF + 8, width), jnp.float32),
                       pltpu.SemaphoreType.DMA,
                       pltpu.SemaphoreType.DMA],
    )
    def k(contrib_h, idx_h, init_h, out_h, idx_v, rows_v, acc_sh, lsem, ssem):
        ci = lax.axis_index("c")
        si = lax.axis_index("s")
        base = ci * HALF
        r0 = si * rows_per_tile

        def lstart(c, b):
            gb = si * groups_per_tile + c * ngc
            pltpu.make_async_copy(idx_h.at[pl.ds(gb, ngc)],
                                  idx_v.at[b], lsem).start()
            pltpu.make_async_copy(contrib_h.at[pl.ds(gb * 128, ngc * 128)],
                                  rows_v.at[b], lsem).start()

        def lwait(c, b):
            gb = si * groups_per_tile + c * ngc
            pltpu.make_async_copy(idx_h.at[pl.ds(gb, ngc)],
                                  idx_v.at[b], lsem).wait()
            pltpu.make_async_copy(contrib_h.at[pl.ds(gb * 128, ngc * 128)],
                                  rows_v.at[b], lsem).wait()

        def swait(b):
            for j in range(ngc):
                pltpu.make_async_copy(rows_v.at[b, pl.ds(j * 128, 128)],
                                      acc_sh.at[idx_v.at[b, j]], ssem).wait()

        pltpu.sync_copy(init_h.at[pl.ds(base + r0, rows_per_tile)],
                        acc_sh.at[pl.ds(r0, rows_per_tile)])
        plsc.subcore_barrier()
        lstart(0, 0)

        def body(c2, carry):
            for b in range(2):
                c = 2 * c2 + b
                lwait(c, b)
                for j in range(ngc):
                    for t in range(8):
                        v = idx_v[b, j, pl.ds(t * 16, 16)] - base
                        ok = (v >= 0) & (v < HALF)
                        idx_v[b, j, pl.ds(t * 16, 16)] = jnp.where(
                            ok, v, jnp.full((16,), trash, jnp.int32))

                @pl.when(c >= 1)
                def _():
                    swait(1 - b)

                @pl.when(c + 1 < chunks)
                def _():
                    lstart(c + 1, 1 - b)

                for j in range(ngc):
                    pltpu.async_copy(rows_v.at[b, pl.ds(j * 128, 128)],
                                     acc_sh.at[idx_v.at[b, j]], ssem,
                                     add=True)
            return carry

        lax.fori_loop(0, chunks // 2, body, 0)
        swait((chunks - 1) % 2)
        plsc.subcore_barrier()
        pltpu.sync_copy(acc_sh.at[pl.ds(r0, rows_per_tile)],
                        out_h.at[pl.ds(base + r0, rows_per_tile)])

    return k(contrib, idx2d, init)


def kernel(x, edge_index, edge_attr, atom_table, bond_table, W_lin, att_src,
           att_dst, att_edge, W_edge, gat_bias, bn_g, bn_b, ln_g, ln_b,
           eu_W1, eu_b1, eu_W2, eu_b2):
    f32 = jnp.float32
    xp = jnp.pad(x, ((0, NP_ - N), (0, 0)))
    eap = jnp.pad(edge_attr, ((0, EP_ - E), (0, 0)))
    src = edge_index[0]
    dst = edge_index[1]
    srcp = jnp.pad(src, (0, EP_ - E)).reshape(EP_ // 128, 128)
    dstg = jnp.pad(dst, (0, EP_ - E)).reshape(EP_ // 128, 128)
    dsts = jnp.pad(dst, (0, EP_ - E),
                   constant_values=np.int32(1 << 30)).reshape(EP_ // 128, 128)
    del f32

    node = _encode(xp, atom_table, _AV, _AOFF, N, NP_, BN)
    edge = _encode(eap, bond_table, _BV, _BOFF, E, EP_, BE)

    for l in range(L):
        asf = att_src[l].reshape(1, D)
        adf = att_dst[l].reshape(1, D)
        aef = att_edge[l].reshape(1, D)
        we = (W_edge[l] * aef) @ jnp.asarray(_RT)
        a_e, esum8 = _edge_attn_pass(edge, we)
        xh, tbl, xha, xhb, xhc = _node_stage(node, W_lin[l], asf, adf, we,
                                             esum8)
        p1 = _sc_gather(tbl, srcp, 16)
        p2 = _sc_gather(tbl, dstg, 16)
        gxh = _sc_gather(xh, srcp, D)
        ca, cb, cc = _edge_ex_pass(p1, p2, a_e, gxh)
        agga = _sc_scatter_add(ca, dsts, xha, 24)
        aggb = _sc_scatter_add(cb, dsts, xhb, 24)
        aggc = _sc_scatter_add(cc, dsts, xhc, 24)
        bias = gat_bias[l].reshape(1, D)
        g = bn_g[l].reshape(1, D)
        b = bn_b[l].reshape(1, D)
        if l < L - 1:
            node, ns_tbl, nd_tbl = _node_finalize(
                agga, aggb, aggc, node, bias, g, b,
                eu_W1[l][0:D], eu_W1[l][D:2 * D], eu_b1[l].reshape(1, D))
            gs = _sc_gather(ns_tbl, srcp, D)
            gd = _sc_gather(nd_tbl, dstg, D)
            edge = _edge_update(gs, gd, edge, eu_W1[l][2 * D:3 * D],
                                eu_W2[l], eu_b2[l].reshape(1, D),
                                ln_g[l].reshape(1, D), ln_b[l].reshape(1, D))
        else:
            node = _node_finalize(agga, aggb, aggc, node, bias, g, b)
    return node[:N]


# speedup vs baseline: 16.6717x; 1.0740x over previous
import functools

import jax
import jax.numpy as jnp
import numpy as np
from jax import lax
from jax.experimental import pallas as pl
from jax.experimental.pallas import tpu as pltpu
from jax.experimental.pallas import tpu_sc as plsc

N = 50000
E = 800000
D = 64
H = 4
C = 16
L = 4
NEG = 0.2
_ATOM_DIMS = [119, 4, 12, 12, 10, 6, 6, 2, 2]
_BOND_DIMS = [5, 6, 2]
_AOFF = np.concatenate([[0], np.cumsum(_ATOM_DIMS)[:-1]]).astype(np.int32)
_BOFF = np.concatenate([[0], np.cumsum(_BOND_DIMS)[:-1]]).astype(np.int32)
_AV = int(sum(_ATOM_DIMS))
_BV = int(sum(_BOND_DIMS))

NP_ = 50048
EP_ = 819200
BN = 2176
BE = 4096
NC, NS = 2, 16
HALF = NP_ // NC
TRASH = HALF

_R = np.zeros((H, D), np.float32)
for h in range(H):
    _R[h, h * C:(h + 1) * C] = 1.0
_RT = _R.T.copy()
_P = np.zeros((3, H, 16), np.float32)
for j in range(3):
    for h in range(H):
        _P[j, h, j * H + h] = 1.0
_S = np.zeros((3, 16, H), np.float32)
for j in range(3):
    for h in range(H):
        _S[j, j * H + h, h] = 1.0
_E64 = np.eye(64, dtype=np.float32)
_SA = _E64[:, 0:24].copy()
_SB = _E64[:, 24:48].copy()
_SC = _E64[:, 48:64].copy()
_RA = _R[:, 0:24].copy()
_RB = _R[:, 24:48].copy()
_RC = _R[:, 48:64].copy()
_E16TO24 = np.zeros((16, 24), np.float32)
_E16TO24[:16, :16] = np.eye(16)
_P4TO24 = np.zeros((4, 24), np.float32)
for h in range(H):
    _P4TO24[h, 16 + h] = 1.0
_SCI = np.zeros((64, 24), np.float32)
_SCI[48:, :16] = np.eye(16)
_ONESC = np.zeros((1, 24), np.float32)
_ONESC[0, 16:20] = 1.0
_TA = _SA.T.copy()
_TB = _SB.T.copy()
_TCM = np.zeros((24, 64), np.float32)
_TCM[:16, 48:] = np.eye(16)
_TD = np.zeros((24, 4), np.float32)
for h in range(H):
    _TD[16 + h, h] = 1.0


def _full(shape):
    nd = len(shape)
    return pl.BlockSpec(shape, lambda i, _n=nd: (0,) * _n)


def _enc_body(nv, offs, nvalid, blk, x_ref, tab_ref, out_ref):
    i = pl.program_id(0)
    xb = x_ref[...]
    oht = jnp.zeros((nv, blk), jnp.float32)
    iot = lax.broadcasted_iota(jnp.int32, (nv, blk), 0)
    for k in range(len(offs)):
        row = lax.slice(xb, (k, 0), (k + 1, blk)) + np.int32(offs[k])
        oht = oht + (row == iot).astype(jnp.float32)
    node = lax.dot_general(oht, tab_ref[...], (((0,), (0,)), ((), ())),
                           preferred_element_type=jnp.float32)
    rid = i * blk + lax.broadcasted_iota(jnp.int32, (blk, 1), 0)
    out_ref[...] = jnp.where(rid < nvalid, node, 0.0)


def _encode(x_t, table, nvals, offs, nvalid, rows, blk):
    nf = x_t.shape[0]
    body = functools.partial(_enc_body, nvals, offs, nvalid, blk)
    return pl.pallas_call(
        body,
        grid=(rows // blk,),
        in_specs=[pl.BlockSpec((nf, blk), lambda i: (0, i)),
                  _full(table.shape)],
        out_specs=pl.BlockSpec((blk, D), lambda i: (i, 0)),
        out_shape=jax.ShapeDtypeStruct((rows, D), jnp.float32),
    )(x_t, table)


def _attn_pass_body(e_ref, we_ref, ae_ref, esum_ref):
    i = pl.program_id(0)
    eb = e_ref[...]
    ae_ref[...] = jnp.dot(eb, we_ref[...], preferred_element_type=jnp.float32)
    ones8 = jnp.ones((8, BE), jnp.float32)
    part = jnp.dot(ones8, eb, preferred_element_type=jnp.float32)

    @pl.when(i == 0)
    def _():
        esum_ref[...] = jnp.zeros_like(esum_ref)

    esum_ref[...] += part


def _edge_attn_pass(edge, we):
    return pl.pallas_call(
        _attn_pass_body,
        grid=(EP_ // BE,),
        in_specs=[pl.BlockSpec((BE, D), lambda i: (i, 0)), _full((D, H))],
        out_specs=[pl.BlockSpec((BE, H), lambda i: (i, 0)),
                   pl.BlockSpec((8, D), lambda i: (0, 0))],
        out_shape=[jax.ShapeDtypeStruct((EP_, H), jnp.float32),
                   jax.ShapeDtypeStruct((8, D), jnp.float32)],
    )(edge, we)


def _node_stage_body(node_ref, wl_ref, asf_ref, adf_ref, we_ref, esum_ref,
                     rt_ref, p_ref, sa_ref, sb_ref, sci_ref, onesc_ref,
                     xh_ref, tbl_ref, xha_ref, xhb_ref, xhc_ref):
    xh = jnp.dot(node_ref[...], wl_ref[...], preferred_element_type=jnp.float32)
    rt = rt_ref[...]
    a_s = jnp.dot(xh * asf_ref[...], rt, preferred_element_type=jnp.float32)
    a_d = jnp.dot(xh * adf_ref[...], rt, preferred_element_type=jnp.float32)
    e_mean = jnp.sum(esum_ref[...], 0, keepdims=True) * np.float32(1.0 / (8 * E))
    a_el = jnp.dot(e_mean, we_ref[...], preferred_element_type=jnp.float32)
    pre = a_s + a_d + a_el
    sub = jnp.where(pre >= 0, pre, pre * np.float32(NEG))
    tbl = (jnp.dot(a_s, p_ref[0], preferred_element_type=jnp.float32)
           + jnp.dot(a_d, p_ref[1], preferred_element_type=jnp.float32)
           + jnp.dot(sub, p_ref[2], preferred_element_type=jnp.float32))
    xh_ref[...] = xh
    tbl_ref[...] = tbl
    xha_ref[...] = jnp.dot(xh, sa_ref[...], preferred_element_type=jnp.float32)
    xhb_ref[...] = jnp.dot(xh, sb_ref[...], preferred_element_type=jnp.float32)
    xhc_ref[...] = jnp.dot(xh, sci_ref[...],
                           preferred_element_type=jnp.float32) + onesc_ref[...]


def _node_stage(node, w_lin, asf, adf, we, esum8):
    return pl.pallas_call(
        _node_stage_body,
        grid=(NP_ // BN,),
        in_specs=[pl.BlockSpec((BN, D), lambda i: (i, 0)), _full((D, D)),
                  _full((1, D)), _full((1, D)), _full((D, H)), _full((8, D)),
                  _full((D, H)), _full((3, H, 16)),
                  _full((D, 24)), _full((D, 24)), _full((D, 24)),
                  _full((1, 24))],
        out_specs=[pl.BlockSpec((BN, D), lambda i: (i, 0)),
                   pl.BlockSpec((BN, 16), lambda i: (i, 0)),
                   pl.BlockSpec((BN, 24), lambda i: (i, 0)),
                   pl.BlockSpec((BN, 24), lambda i: (i, 0)),
                   pl.BlockSpec((BN, 24), lambda i: (i, 0))],
        out_shape=[jax.ShapeDtypeStruct((NP_, D), jnp.float32),
                   jax.ShapeDtypeStruct((NP_, 16), jnp.float32),
                   jax.ShapeDtypeStruct((NP_, 24), jnp.float32),
                   jax.ShapeDtypeStruct((NP_, 24), jnp.float32),
                   jax.ShapeDtypeStruct((NP_, 24), jnp.float32)],
    )(node, w_lin, asf, adf, we, esum8, jnp.asarray(_RT), jnp.asarray(_P),
      jnp.asarray(_SA), jnp.asarray(_SB), jnp.asarray(_SCI),
      jnp.asarray(_ONESC))


def _edge_ex_body(p1_ref, p2_ref, ae_ref, gxh_ref, s_ref, ra_ref,
                  rb_ref, rc_ref, sa_ref, sb_ref, sc_ref, e24_ref, p24_ref,
                  ca_ref, cb_ref, cc_ref):
    p1 = p1_ref[...]
    p2 = p2_ref[...]
    a = (jnp.dot(p1, s_ref[0], preferred_element_type=jnp.float32)
         + jnp.dot(p2, s_ref[1], preferred_element_type=jnp.float32)
         + ae_ref[...])
    a = jnp.where(a >= 0, a, a * np.float32(NEG))
    sub = jnp.dot(p2, s_ref[2], preferred_element_type=jnp.float32)
    ex = jnp.exp(a - sub)
    gxh = gxh_ref[...]
    ca_ref[...] = jnp.dot(gxh, sa_ref[...],
                          preferred_element_type=jnp.float32) * jnp.dot(
        ex, ra_ref[...], preferred_element_type=jnp.float32)
    cb_ref[...] = jnp.dot(gxh, sb_ref[...],
                          preferred_element_type=jnp.float32) * jnp.dot(
        ex, rb_ref[...], preferred_element_type=jnp.float32)
    cpart = jnp.dot(gxh, sc_ref[...],
                    preferred_element_type=jnp.float32) * jnp.dot(
        ex, rc_ref[...], preferred_element_type=jnp.float32)
    cc_ref[...] = (jnp.dot(cpart, e24_ref[...],
                           preferred_element_type=jnp.float32)
                   + jnp.dot(ex, p24_ref[...],
                             preferred_element_type=jnp.float32))


def _edge_ex_pass(p1, p2, a_e, gxh):
    return pl.pallas_call(
        _edge_ex_body,
        grid=(EP_ // BE,),
        in_specs=[pl.BlockSpec((BE, 16), lambda i: (i, 0)),
                  pl.BlockSpec((BE, 16), lambda i: (i, 0)),
                  pl.BlockSpec((BE, H), lambda i: (i, 0)),
                  pl.BlockSpec((BE, D), lambda i: (i, 0)),
                  _full((3, 16, H)), _full((H, 24)), _full((H, 24)),
                  _full((H, 16)), _full((D, 24)), _full((D, 24)),
                  _full((D, 16)), _full((16, 24)), _full((H, 24))],
        out_specs=[pl.BlockSpec((BE, 24), lambda i: (i, 0)),
                   pl.BlockSpec((BE, 24), lambda i: (i, 0)),
                   pl.BlockSpec((BE, 24), lambda i: (i, 0))],
        out_shape=[jax.ShapeDtypeStruct((EP_, 24), jnp.float32),
                   jax.ShapeDtypeStruct((EP_, 24), jnp.float32),
                   jax.ShapeDtypeStruct((EP_, 24), jnp.float32)],
    )(p1, p2, a_e, gxh, jnp.asarray(_S), jnp.asarray(_RA),
      jnp.asarray(_RB), jnp.asarray(_RC), jnp.asarray(_SA),
      jnp.asarray(_SB), jnp.asarray(_SC), jnp.asarray(_E16TO24),
      jnp.asarray(_P4TO24))


def _ln(x, g, b):
    mu = jnp.mean(x, -1, keepdims=True)
    xc = x - mu
    var = jnp.mean(xc * xc, -1, keepdims=True)
    return xc / jnp.sqrt(var + np.float32(1e-5)) * g + b


def _finalize_body(eu, agga_ref, aggb_ref, aggc_ref, node_ref, bias_ref,
                   g_ref, b_ref, ta_ref, tb_ref, tcm_ref, td_ref, r_ref,
                   *rest):
    aggc = aggc_ref[...]
    invd = 1.0 / (jnp.dot(aggc, td_ref[...],
                          preferred_element_type=jnp.float32) + np.float32(1e-16))
    aggsum = (jnp.dot(agga_ref[...], ta_ref[...],
                      preferred_element_type=jnp.float32)
              + jnp.dot(aggb_ref[...], tb_ref[...],
                        preferred_element_type=jnp.float32)
              + jnp.dot(aggc, tcm_ref[...],
                        preferred_element_type=jnp.float32))
    agg = aggsum * jnp.dot(invd, r_ref[...],
                           preferred_element_type=jnp.float32)
    conv = _ln(agg + bias_ref[...], g_ref[...], b_ref[...])
    nn = jnp.maximum(conv, 0.0) + node_ref[...]
    if eu:
        w1a_ref, w1b_ref, b1_ref, out_ref, ns_ref, nd_ref = rest
        out_ref[...] = nn
        ns_ref[...] = jnp.dot(nn, w1a_ref[...],
                              preferred_element_type=jnp.float32) + b1_ref[...]
        nd_ref[...] = jnp.dot(nn, w1b_ref[...],
                              preferred_element_type=jnp.float32)
    else:
        rest[0][...] = nn


def _node_finalize(agga, aggb, aggc, node, bias, g, b,
                   w1a=None, w1b=None, b1=None):
    eu = w1a is not None
    ins = [agga, aggb, aggc, node, bias, g, b, jnp.asarray(_TA),
           jnp.asarray(_TB), jnp.asarray(_TCM), jnp.asarray(_TD),
           jnp.asarray(_R)]
    in_specs = [pl.BlockSpec((BN, 24), lambda i: (i, 0)),
                pl.BlockSpec((BN, 24), lambda i: (i, 0)),
                pl.BlockSpec((BN, 24), lambda i: (i, 0)),
                pl.BlockSpec((BN, D), lambda i: (i, 0)),
                _full((1, D)), _full((1, D)), _full((1, D)),
                _full((24, D)), _full((24, D)), _full((24, D)),
                _full((24, H)), _full((H, D))]
    nout = 3 if eu else 1
    if eu:
        ins += [w1a, w1b, b1]
        in_specs += [_full((D, D)), _full((D, D)), _full((1, D))]
    out = pl.pallas_call(
        functools.partial(_finalize_body, eu),
        grid=(NP_ // BN,),
        in_specs=in_specs,
        out_specs=[pl.BlockSpec((BN, D), lambda i: (i, 0))] * nout,
        out_shape=[jax.ShapeDtypeStruct((NP_, D), jnp.float32)] * nout,
    )(*ins)
    return out if eu else out[0]


def _edge_up_body(gs_ref, gd_ref, e_ref, w1c_ref, w2_ref, b2_ref, g_ref,
                  b_ref, out_ref):
    i = pl.program_id(0)
    eb = e_ref[...]
    h = gs_ref[...] + gd_ref[...] + jnp.dot(
        eb, w1c_ref[...], preferred_element_type=jnp.float32)
    h = jnp.maximum(h, 0.0)
    h = jnp.dot(h, w2_ref[...], preferred_element_type=jnp.float32) + b2_ref[...]
    h = _ln(h, g_ref[...], b_ref[...])
    en = jnp.maximum(h, 0.0) + eb
    rid = i * BE + lax.broadcasted_iota(jnp.int32, (BE, 1), 0)
    out_ref[...] = jnp.where(rid < E, en, 0.0)


def _edge_update(gs, gd, edge, w1c, w2, b2, g, b):
    return pl.pallas_call(
        _edge_up_body,
        grid=(EP_ // BE,),
        in_specs=[pl.BlockSpec((BE, D), lambda i: (i, 0)),
                  pl.BlockSpec((BE, D), lambda i: (i, 0)),
                  pl.BlockSpec((BE, D), lambda i: (i, 0)),
                  _full((D, D)), _full((D, D)), _full((1, D)),
                  _full((1, D)), _full((1, D))],
        out_specs=pl.BlockSpec((BE, D), lambda i: (i, 0)),
        out_shape=jax.ShapeDtypeStruct((EP_, D), jnp.float32),
    )(gs, gd, edge, w1c, w2, b2, g, b)


def _sc_gather(table, idx2d, width):
    ngc = 5 if width >= 64 else 10
    groups_per_tile = EP_ // 128 // (NC * NS)
    chunks = groups_per_tile // ngc
    mesh = plsc.VectorSubcoreMesh(core_axis_name="c", subcore_axis_name="s")

    @functools.partial(
        pl.kernel, mesh=mesh,
        out_type=jax.ShapeDtypeStruct((EP_, width), jnp.float32),
        compiler_params=pltpu.CompilerParams(use_tc_tiling_on_sc=False),
        scratch_types=[pltpu.VMEM((2, ngc, 128), jnp.int32),
                       pltpu.VMEM((2, ngc * 128, width), jnp.float32),
                       pltpu.SemaphoreType.DMA,
                       pltpu.SemaphoreType.DMA,
                       pltpu.SemaphoreType.DMA],
    )
    def k(table_h, idx_h, out_h, idx_v, rows_v, isem, gsem, osem):
        wid = lax.axis_index("s") * NC + lax.axis_index("c")
        g0 = wid * groups_per_tile

        def idx_cp(c, b):
            return pltpu.make_async_copy(
                idx_h.at[pl.ds(g0 + c * ngc, ngc)], idx_v.at[b], isem)

        def out_cp(c, b):
            return pltpu.make_async_copy(
                rows_v.at[b],
                out_h.at[pl.ds((g0 + c * ngc) * 128, ngc * 128)], osem)

        idx_cp(0, 0).start()

        def body(c2, carry):
            for b in range(2):
                c = 2 * c2 + b
                idx_cp(c, b).wait()

                @pl.when(c + 1 < chunks)
                def _():
                    idx_cp(c + 1, 1 - b).start()

                @pl.when(c >= 2)
                def _():
                    out_cp(c - 2, b).wait()

                cps = [pltpu.async_copy(table_h.at[idx_v.at[b, j]],
                                        rows_v.at[b, pl.ds(j * 128, 128)],
                                        gsem)
                       for j in range(ngc)]
                for cp in cps:
                    cp.wait()
                out_cp(c, b).start()
            return carry

        lax.fori_loop(0, chunks // 2, body, 0)
        out_cp(chunks - 2, 0).wait()
        out_cp(chunks - 1, 1).wait()

    return k(table, idx2d)


def _sc_gather_pair(table, idxa, idxb):
    width, ngc = 16, 10
    groups_per_tile = EP_ // 128 // (NC * NS)
    chunks = groups_per_tile // ngc
    mesh = plsc.VectorSubcoreMesh(core_axis_name="c", subcore_axis_name="s")

    @functools.partial(
        pl.kernel, mesh=mesh,
        out_type=[jax.ShapeDtypeStruct((EP_, width), jnp.float32),
                  jax.ShapeDtypeStruct((EP_, width), jnp.float32)],
        compiler_params=pltpu.CompilerParams(use_tc_tiling_on_sc=False),
        scratch_types=[pltpu.VMEM((2, 2, ngc, 128), jnp.int32),
                       pltpu.VMEM((2, 2, ngc * 128, width), jnp.float32),
                       pltpu.SemaphoreType.DMA,
                       pltpu.SemaphoreType.DMA,
                       pltpu.SemaphoreType.DMA],
    )
    def k(table_h, idxa_h, idxb_h, outa_h, outb_h, idx_v, rows_v, isem,
          gsem, osem):
        wid = lax.axis_index("s") * NC + lax.axis_index("c")
        g0 = wid * groups_per_tile
        idx_hs = [idxa_h, idxb_h]
        out_hs = [outa_h, outb_h]

        def idx_cp(c, b, w):
            return pltpu.make_async_copy(
                idx_hs[w].at[pl.ds(g0 + c * ngc, ngc)], idx_v.at[b, w], isem)

        def out_cp(c, b, w):
            return pltpu.make_async_copy(
                rows_v.at[b, w],
                out_hs[w].at[pl.ds((g0 + c * ngc) * 128, ngc * 128)], osem)

        idx_cp(0, 0, 0).start()
        idx_cp(0, 0, 1).start()

        def body(c2, carry):
            for b in range(2):
                c = 2 * c2 + b
                for w in range(2):
                    idx_cp(c, b, w).wait()

                @pl.when(c + 1 < chunks)
                def _():
                    idx_cp(c + 1, 1 - b, 0).start()
                    idx_cp(c + 1, 1 - b, 1).start()

                @pl.when(c >= 2)
                def _():
                    out_cp(c - 2, b, 0).wait()
                    out_cp(c - 2, b, 1).wait()

                cps = [pltpu.async_copy(table_h.at[idx_v.at[b, w, j]],
                                        rows_v.at[b, w, pl.ds(j * 128, 128)],
                                        gsem)
                       for w in range(2) for j in range(ngc)]
                for cp in cps:
                    cp.wait()
                out_cp(c, b, 0).start()
                out_cp(c, b, 1).start()
            return carry

        lax.fori_loop(0, chunks // 2, body, 0)
        for w in range(2):
            out_cp(chunks - 2, 0, w).wait()
            out_cp(chunks - 1, 1, w).wait()

    return k(table, idxa, idxb)


def _sc_scatter_add(contrib, idx2d, init, width):
    ngc = 10
    groups_per_tile = EP_ // 128 // NS
    chunks = groups_per_tile // ngc
    trash = HALF
    rows_per_tile = HALF // NS
    mesh = plsc.VectorSubcoreMesh(core_axis_name="c", subcore_axis_name="s")

    @functools.partial(
        pl.kernel, mesh=mesh,
        out_type=jax.ShapeDtypeStruct((NP_, width), jnp.float32),
        compiler_params=pltpu.CompilerParams(use_tc_tiling_on_sc=False),
        scratch_types=[pltpu.VMEM((2, ngc, 128), jnp.int32),
                       pltpu.VMEM((2, ngc * 128, width), jnp.float32),
                       pltpu.VMEM_SHARED((HALF + 8, width), jnp.float32),
                       pltpu.SemaphoreType.DMA,
                       pltpu.SemaphoreType.DMA],
    )
    def k(contrib_h, idx_h, init_h, out_h, idx_v, rows_v, acc_sh, lsem, ssem):
        ci = lax.axis_index("c")
        si = lax.axis_index("s")
        base = ci * HALF
        r0 = si * rows_per_tile

        def lstart(c, b):
            gb = si * groups_per_tile + c * ngc
            pltpu.make_async_copy(idx_h.at[pl.ds(gb, ngc)],
                                  idx_v.at[b], lsem).start()
            pltpu.make_async_copy(contrib_h.at[pl.ds(gb * 128, ngc * 128)],
                                  rows_v.at[b], lsem).start()

        def lwait(c, b):
            gb = si * groups_per_tile + c * ngc
            pltpu.make_async_copy(idx_h.at[pl.ds(gb, ngc)],
                                  idx_v.at[b], lsem).wait()
            pltpu.make_async_copy(contrib_h.at[pl.ds(gb * 128, ngc * 128)],
                                  rows_v.at[b], lsem).wait()

        def swait(b):
            for j in range(ngc):
                pltpu.make_async_copy(rows_v.at[b, pl.ds(j * 128, 128)],
                                      acc_sh.at[idx_v.at[b, j]], ssem).wait()

        pltpu.sync_copy(init_h.at[pl.ds(base + r0, rows_per_tile)],
                        acc_sh.at[pl.ds(r0, rows_per_tile)])
        plsc.subcore_barrier()
        lstart(0, 0)

        def body(c2, carry):
            for b in range(2):
                c = 2 * c2 + b
                lwait(c, b)
                for j in range(ngc):
                    for t in range(8):
                        v = idx_v[b, j, pl.ds(t * 16, 16)] - base
                        ok = (v >= 0) & (v < HALF)
                        idx_v[b, j, pl.ds(t * 16, 16)] = jnp.where(
                            ok, v, jnp.full((16,), trash, jnp.int32))

                @pl.when(c >= 1)
                def _():
                    swait(1 - b)

                @pl.when(c + 1 < chunks)
                def _():
                    lstart(c + 1, 1 - b)

                for j in range(ngc):
                    pltpu.async_copy(rows_v.at[b, pl.ds(j * 128, 128)],
                                     acc_sh.at[idx_v.at[b, j]], ssem,
                                     add=True)
            return carry

        lax.fori_loop(0, chunks // 2, body, 0)
        swait((chunks - 1) % 2)
        plsc.subcore_barrier()
        pltpu.sync_copy(acc_sh.at[pl.ds(r0, rows_per_tile)],
                        out_h.at[pl.ds(base + r0, rows_per_tile)])

    return k(contrib, idx2d, init)


def kernel(x, edge_index, edge_attr, atom_table, bond_table, W_lin, att_src,
           att_dst, att_edge, W_edge, gat_bias, bn_g, bn_b, ln_g, ln_b,
           eu_W1, eu_b1, eu_W2, eu_b2):
    xp_t = jnp.pad(x.T, ((0, 0), (0, NP_ - N)))
    eap_t = jnp.pad(edge_attr.T, ((0, 0), (0, EP_ - E)))
    src = edge_index[0]
    dst = edge_index[1]
    srcp = jnp.pad(src, (0, EP_ - E)).reshape(EP_ // 128, 128)
    dstg = jnp.pad(dst, (0, EP_ - E)).reshape(EP_ // 128, 128)
    dsts = jnp.pad(dst, (0, EP_ - E),
                   constant_values=np.int32(1 << 30)).reshape(EP_ // 128, 128)

    node = _encode(xp_t, atom_table, _AV, _AOFF, N, NP_, BN)
    edge = _encode(eap_t, bond_table, _BV, _BOFF, E, EP_, BE)

    for l in range(L):
        asf = att_src[l].reshape(1, D)
        adf = att_dst[l].reshape(1, D)
        aef = att_edge[l].reshape(1, D)
        we = (W_edge[l] * aef) @ jnp.asarray(_RT)
        a_e, esum8 = _edge_attn_pass(edge, we)
        xh, tbl, xha, xhb, xhc = _node_stage(node, W_lin[l], asf, adf, we,
                                             esum8)
        p1, p2 = _sc_gather_pair(tbl, srcp, dstg)
        gxh = _sc_gather(xh, srcp, D)
        ca, cb, cc = _edge_ex_pass(p1, p2, a_e, gxh)
        agga = _sc_scatter_add(ca, dsts, xha, 24)
        aggb = _sc_scatter_add(cb, dsts, xhb, 24)
        aggc = _sc_scatter_add(cc, dsts, xhc, 24)
        bias = gat_bias[l].reshape(1, D)
        g = bn_g[l].reshape(1, D)
        b = bn_b[l].reshape(1, D)
        if l < L - 1:
            node, ns_tbl, nd_tbl = _node_finalize(
                agga, aggb, aggc, node, bias, g, b,
                eu_W1[l][0:D], eu_W1[l][D:2 * D], eu_b1[l].reshape(1, D))
            gs = _sc_gather(ns_tbl, srcp, D)
            gd = _sc_gather(nd_tbl, dstg, D)
            edge = _edge_update(gs, gd, edge, eu_W1[l][2 * D:3 * D],
                                eu_W2[l], eu_b2[l].reshape(1, D),
                                ln_g[l].reshape(1, D), ln_b[l].reshape(1, D))
        else:
            node = _node_finalize(agga, aggb, aggc, node, bias, g, b)
    return node[:N]


# speedup vs baseline: 17.3053x; 1.0380x over previous
import functools

import jax
import jax.numpy as jnp
import numpy as np
from jax import lax
from jax.experimental import pallas as pl
from jax.experimental.pallas import tpu as pltpu
from jax.experimental.pallas import tpu_sc as plsc

N = 50000
E = 800000
D = 64
H = 4
C = 16
L = 4
NEG = 0.2
_ATOM_DIMS = [119, 4, 12, 12, 10, 6, 6, 2, 2]
_BOND_DIMS = [5, 6, 2]
_AOFF = np.concatenate([[0], np.cumsum(_ATOM_DIMS)[:-1]]).astype(np.int32)
_BOFF = np.concatenate([[0], np.cumsum(_BOND_DIMS)[:-1]]).astype(np.int32)
_AV = int(sum(_ATOM_DIMS))
_BV = int(sum(_BOND_DIMS))

NP_ = 50048
EP_ = 819200
BN = 2176
BE = 4096
NC, NS = 2, 16
HALF = NP_ // NC
_NGC = {16: 10, 72: 4, 80: 4, 144: 2}

_SW = [72, 144, 144, 144]
_XOF = [0, 64, 64, 64]
_ASOF = [64, 128, 128, 128]
_DW = [16, 80, 80, 80]
_ADOF = [0, 64, 64, 64]
_SUBOF = [4, 68, 68, 68]


def _embm(win, wout, off):
    m = np.zeros((win, wout), np.float32)
    m[np.arange(win), off + np.arange(win)] = 1.0
    return m


def _selm(win, off, wout):
    m = np.zeros((win, wout), np.float32)
    m[off + np.arange(wout), np.arange(wout)] = 1.0
    return m


_R = np.zeros((H, D), np.float32)
for h in range(H):
    _R[h, h * C:(h + 1) * C] = 1.0
_RT = _R.T.copy()
_E64 = np.eye(64, dtype=np.float32)
_SA = _E64[:, 0:24].copy()
_SB = _E64[:, 24:48].copy()
_SC = _E64[:, 48:64].copy()
_RA = _R[:, 0:24].copy()
_RB = _R[:, 24:48].copy()
_RC = _R[:, 48:64].copy()
_E16TO24 = np.zeros((16, 24), np.float32)
_E16TO24[:16, :16] = np.eye(16)
_P4TO24 = _embm(4, 24, 16)
_TA = _SA.T.copy()
_TB = _SB.T.copy()
_TCM = np.zeros((24, 64), np.float32)
_TCM[:16, 48:] = np.eye(16)
_TD = np.zeros((24, 4), np.float32)
for h in range(H):
    _TD[16 + h, h] = 1.0

_LSEL = []
for l in range(L):
    sw, dw = _SW[l], _DW[l]
    xa = np.zeros((sw, 24), np.float32)
    xa[_XOF[l]:_XOF[l] + 64, :] = _SA
    xb = np.zeros((sw, 24), np.float32)
    xb[_XOF[l]:_XOF[l] + 64, :] = _SB
    xc = np.zeros((sw, 16), np.float32)
    xc[_XOF[l]:_XOF[l] + 64, :] = _SC
    sel = dict(AS=_selm(sw, _ASOF[l], 4), AD=_selm(dw, _ADOF[l], 4),
               SUB=_selm(dw, _SUBOF[l], 4), XA=xa, XB=xb, XC=xc)
    if l > 0:
        sel["GS"] = _selm(sw, 0, 64)
        sel["GD"] = _selm(dw, 0, 64)
    _LSEL.append(sel)


def _full(shape):
    nd = len(shape)
    return pl.BlockSpec(shape, lambda i, _n=nd: (0,) * _n)


def _leaky(x):
    return jnp.where(x >= 0, x, x * np.float32(NEG))


def _dot(a, b):
    return jnp.dot(a, b, preferred_element_type=jnp.float32)


def _ln(x, g, b):
    mu = jnp.mean(x, -1, keepdims=True)
    xc = x - mu
    var = jnp.mean(xc * xc, -1, keepdims=True)
    return xc / jnp.sqrt(var + np.float32(1e-5)) * g + b


def _enc_body(nv, offs, nvalid, blk, x_ref, tab_ref, out_ref):
    i = pl.program_id(0)
    xb = x_ref[...]
    oht = jnp.zeros((nv, blk), jnp.float32)
    iot = lax.broadcasted_iota(jnp.int32, (nv, blk), 0)
    for k in range(len(offs)):
        row = lax.slice(xb, (k, 0), (k + 1, blk)) + np.int32(offs[k])
        oht = oht + (row == iot).astype(jnp.float32)
    node = lax.dot_general(oht, tab_ref[...], (((0,), (0,)), ((), ())),
                           preferred_element_type=jnp.float32)
    rid = i * blk + lax.broadcasted_iota(jnp.int32, (blk, 1), 0)
    out_ref[...] = jnp.where(rid < nvalid, node, 0.0)


def _encode(x_t, table, nvals, offs, nvalid, rows, blk):
    nf = x_t.shape[0]
    body = functools.partial(_enc_body, nvals, offs, nvalid, blk)
    return pl.pallas_call(
        body,
        grid=(rows // blk,),
        in_specs=[pl.BlockSpec((nf, blk), lambda i: (0, i)),
                  _full(table.shape)],
        out_specs=pl.BlockSpec((blk, D), lambda i: (i, 0)),
        out_shape=jax.ShapeDtypeStruct((rows, D), jnp.float32),
    )(x_t, table)


def _attn_pass_body(e_ref, we_ref, ae_ref, esum_ref):
    i = pl.program_id(0)
    eb = e_ref[...]
    ae_ref[...] = _dot(eb, we_ref[...])
    part = _dot(jnp.ones((8, BE), jnp.float32), eb)

    @pl.when(i == 0)
    def _():
        esum_ref[...] = jnp.zeros_like(esum_ref)

    esum_ref[...] += part


def _edge_attn_pass(edge, we):
    return pl.pallas_call(
        _attn_pass_body,
        grid=(EP_ // BE,),
        in_specs=[pl.BlockSpec((BE, D), lambda i: (i, 0)), _full((D, H))],
        out_specs=[pl.BlockSpec((BE, H), lambda i: (i, 0)),
                   pl.BlockSpec((8, D), lambda i: (0, 0))],
        out_shape=[jax.ShapeDtypeStruct((EP_, H), jnp.float32),
                   jax.ShapeDtypeStruct((8, D), jnp.float32)],
    )(edge, we)


def _stage0_body(node_ref, wl_ref, asf_ref, adf_ref, rt_ref, ex_ref,
                 eas_ref, ead_ref, esub_ref, ts_ref, td_ref):
    xh = _dot(node_ref[...], wl_ref[...])
    rt = rt_ref[...]
    a_s = _dot(xh * asf_ref[...], rt)
    a_d = _dot(xh * adf_ref[...], rt)
    sub = _leaky(a_s + a_d)
    ts_ref[...] = _dot(xh, ex_ref[...]) + _dot(a_s, eas_ref[...])
    td_ref[...] = _dot(a_d, ead_ref[...]) + _dot(sub, esub_ref[...])


def _node_stage0(node, w_lin, asf, adf):
    return pl.pallas_call(
        _stage0_body,
        grid=(NP_ // BN,),
        in_specs=[pl.BlockSpec((BN, D), lambda i: (i, 0)), _full((D, D)),
                  _full((1, D)), _full((1, D)), _full((D, H)),
                  _full((D, 72)), _full((H, 72)), _full((H, 16)),
                  _full((H, 16))],
        out_specs=[pl.BlockSpec((BN, 72), lambda i: (i, 0)),
                   pl.BlockSpec((BN, 16), lambda i: (i, 0))],
        out_shape=[jax.ShapeDtypeStruct((NP_, 72), jnp.float32),
                   jax.ShapeDtypeStruct((NP_, 16), jnp.float32)],
    )(node, w_lin, asf, adf, jnp.asarray(_RT), jnp.asarray(_embm(64, 72, 0)),
      jnp.asarray(_embm(4, 72, 64)), jnp.asarray(_embm(4, 16, 0)),
      jnp.asarray(_embm(4, 16, 4)))


def _selmats(l):
    sel = _LSEL[l]
    mats = [sel["AS"], sel["AD"], sel["SUB"], sel["XA"], sel["XB"],
            sel["XC"], _RA, _RB, _RC, _E16TO24, _P4TO24]
    return ([jnp.asarray(m) for m in mats], [_full(m.shape) for m in mats])


def _init_body(ts_ref, td_ref, esum_ref, we_ref, as_ref, ad_ref, sub_ref,
               xa_ref, xb_ref, xc_ref, ra_ref, rb_ref, rc_ref, e16_ref,
               p24_ref, ia_ref, ib_ref, ic_ref):
    ts = ts_ref[...]
    td = td_ref[...]
    a_s = _dot(ts, as_ref[...])
    a_d = _dot(td, ad_ref[...])
    sub = _dot(td, sub_ref[...])
    e_mean = jnp.sum(esum_ref[...], 0, keepdims=True) * np.float32(
        1.0 / (8 * E))
    al = _leaky(a_s + a_d + _dot(e_mean, we_ref[...]))
    lex = jnp.exp(al - sub)
    ia_ref[...] = _dot(ts, xa_ref[...]) * _dot(lex, ra_ref[...])
    ib_ref[...] = _dot(ts, xb_ref[...]) * _dot(lex, rb_ref[...])
    cpart = _dot(ts, xc_ref[...]) * _dot(lex, rc_ref[...])
    ic_ref[...] = _dot(cpart, e16_ref[...]) + _dot(lex, p24_ref[...])


def _init_pass(ts, td, esum8, we, l):
    sw, dw = _SW[l], _DW[l]
    cins, cspecs = _selmats(l)
    return pl.pallas_call(
        _init_body,
        grid=(NP_ // BN,),
        in_specs=[pl.BlockSpec((BN, sw), lambda i: (i, 0)),
                  pl.BlockSpec((BN, dw), lambda i: (i, 0)),
                  _full((8, D)), _full((D, H))] + cspecs,
        out_specs=[pl.BlockSpec((BN, 24), lambda i: (i, 0))] * 3,
        out_shape=[jax.ShapeDtypeStruct((NP_, 24), jnp.float32)] * 3,
    )(ts, td, esum8, we, *cins)


def _ex_body(sr_ref, dr_ref, ae_ref, as_ref, ad_ref, sub_ref, xa_ref,
             xb_ref, xc_ref, ra_ref, rb_ref, rc_ref, e16_ref, p24_ref,
             ca_ref, cb_ref, cc_ref):
    sr = sr_ref[...]
    dr = dr_ref[...]
    a = _leaky(_dot(sr, as_ref[...]) + _dot(dr, ad_ref[...]) + ae_ref[...])
    ex = jnp.exp(a - _dot(dr, sub_ref[...]))
    ca_ref[...] = _dot(sr, xa_ref[...]) * _dot(ex, ra_ref[...])
    cb_ref[...] = _dot(sr, xb_ref[...]) * _dot(ex, rb_ref[...])
    cpart = _dot(sr, xc_ref[...]) * _dot(ex, rc_ref[...])
    cc_ref[...] = _dot(cpart, e16_ref[...]) + _dot(ex, p24_ref[...])


def _edge_ex_pass(sr, dr, a_e, l):
    sw, dw = _SW[l], _DW[l]
    cins, cspecs = _selmats(l)
    return pl.pallas_call(
        _ex_body,
        grid=(EP_ // BE,),
        in_specs=[pl.BlockSpec((BE, sw), lambda i: (i, 0)),
                  pl.BlockSpec((BE, dw), lambda i: (i, 0)),
                  pl.BlockSpec((BE, H), lambda i: (i, 0))] + cspecs,
        out_specs=[pl.BlockSpec((BE, 24), lambda i: (i, 0))] * 3,
        out_shape=[jax.ShapeDtypeStruct((EP_, 24), jnp.float32)] * 3,
    )(sr, dr, a_e, *cins)


def _finalize_body(eu, agga_ref, aggb_ref, aggc_ref, node_ref, bias_ref,
                   g_ref, b_ref, ta_ref, tb_ref, tcm_ref, tdm_ref, r_ref,
                   *rest):
    aggc = aggc_ref[...]
    invd = 1.0 / (_dot(aggc, tdm_ref[...]) + np.float32(1e-16))
    aggsum = (_dot(agga_ref[...], ta_ref[...])
              + _dot(aggb_ref[...], tb_ref[...])
              + _dot(aggc, tcm_ref[...]))
    agg = aggsum * _dot(invd, r_ref[...])
    conv = _ln(agg + bias_ref[...], g_ref[...], b_ref[...])
    nn = jnp.maximum(conv, 0.0) + node_ref[...]
    if not eu:
        rest[-1][...] = nn
        return
    (w1a_ref, w1b_ref, b1_ref, wln_ref, asf_ref, adf_ref, rt_ref,
     en_ref, ex_ref, eas_ref, end_ref, ead_ref, esub_ref,
     out_ref, ts_ref, td_ref) = rest
    out_ref[...] = nn
    ns = _dot(nn, w1a_ref[...]) + b1_ref[...]
    nd = _dot(nn, w1b_ref[...])
    xh = _dot(nn, wln_ref[...])
    rt = rt_ref[...]
    a_s = _dot(xh * asf_ref[...], rt)
    a_d = _dot(xh * adf_ref[...], rt)
    sub = _leaky(a_s + a_d)
    ts_ref[...] = (_dot(ns, en_ref[...]) + _dot(xh, ex_ref[...])
                   + _dot(a_s, eas_ref[...]))
    td_ref[...] = (_dot(nd, end_ref[...]) + _dot(a_d, ead_ref[...])
                   + _dot(sub, esub_ref[...]))


def _node_finalize(agga, aggb, aggc, node, bias, g, b, eu_args=None):
    eu = eu_args is not None
    ins = [agga, aggb, aggc, node, bias, g, b, jnp.asarray(_TA),
           jnp.asarray(_TB), jnp.asarray(_TCM), jnp.asarray(_TD),
           jnp.asarray(_R)]
    in_specs = [pl.BlockSpec((BN, 24), lambda i: (i, 0)),
                pl.BlockSpec((BN, 24), lambda i: (i, 0)),
                pl.BlockSpec((BN, 24), lambda i: (i, 0)),
                pl.BlockSpec((BN, D), lambda i: (i, 0)),
                _full((1, D)), _full((1, D)), _full((1, D)),
                _full((24, D)), _full((24, D)), _full((24, D)),
                _full((24, H)), _full((H, D))]
    if eu:
        w1a, w1b, b1, wln, asf, adf = eu_args
        ins += [w1a, w1b, b1, wln, asf, adf, jnp.asarray(_RT),
                jnp.asarray(_embm(64, 144, 0)), jnp.asarray(_embm(64, 144, 64)),
                jnp.asarray(_embm(4, 144, 128)), jnp.asarray(_embm(64, 80, 0)),
                jnp.asarray(_embm(4, 80, 64)), jnp.asarray(_embm(4, 80, 68))]
        in_specs += [_full((D, D)), _full((D, D)), _full((1, D)),
                     _full((D, D)), _full((1, D)), _full((1, D)),
                     _full((D, H)), _full((D, 144)), _full((D, 144)),
                     _full((H, 144)), _full((D, 80)), _full((H, 80)),
                     _full((H, 80))]
        out_specs = [pl.BlockSpec((BN, D), lambda i: (i, 0)),
                     pl.BlockSpec((BN, 144), lambda i: (i, 0)),
                     pl.BlockSpec((BN, 80), lambda i: (i, 0))]
        out_shape = [jax.ShapeDtypeStruct((NP_, D), jnp.float32),
                     jax.ShapeDtypeStruct((NP_, 144), jnp.float32),
                     jax.ShapeDtypeStruct((NP_, 80), jnp.float32)]
    else:
        out_specs = [pl.BlockSpec((BN, D), lambda i: (i, 0))]
        out_shape = [jax.ShapeDtypeStruct((NP_, D), jnp.float32)]
    out = pl.pallas_call(
        functools.partial(_finalize_body, eu),
        grid=(NP_ // BN,),
        in_specs=in_specs,
        out_specs=out_specs,
        out_shape=out_shape,
    )(*ins)
    return out if eu else out[0]


def _edge_up_body(sr_ref, dr_ref, e_ref, w1c_ref, w2_ref, b2_ref,
                  g_ref, b_ref, gs_ref, gd_ref, out_ref):
    i = pl.program_id(0)
    eb = e_ref[...]
    h = (_dot(sr_ref[...], gs_ref[...]) + _dot(dr_ref[...], gd_ref[...])
         + _dot(eb, w1c_ref[...]))
    h = jnp.maximum(h, 0.0)
    h = _dot(h, w2_ref[...]) + b2_ref[...]
    h = _ln(h, g_ref[...], b_ref[...])
    en = jnp.maximum(h, 0.0) + eb
    rid = i * BE + lax.broadcasted_iota(jnp.int32, (BE, 1), 0)
    out_ref[...] = jnp.where(rid < E, en, 0.0)


def _edge_update(sr, dr, edge, w1c, w2, b2, g, b, l):
    sw, dw = _SW[l], _DW[l]
    return pl.pallas_call(
        _edge_up_body,
        grid=(EP_ // BE,),
        in_specs=[pl.BlockSpec((BE, sw), lambda i: (i, 0)),
                  pl.BlockSpec((BE, dw), lambda i: (i, 0)),
                  pl.BlockSpec((BE, D), lambda i: (i, 0)),
                  _full((D, D)), _full((D, D)), _full((1, D)),
                  _full((1, D)), _full((1, D)),
                  _full((sw, D)), _full((dw, D))],
        out_specs=pl.BlockSpec((BE, D), lambda i: (i, 0)),
        out_shape=jax.ShapeDtypeStruct((EP_, D), jnp.float32),
    )(sr, dr, edge, w1c, w2, b2, g, b, jnp.asarray(_LSEL[l]["GS"]),
      jnp.asarray(_LSEL[l]["GD"]))


def _sc_gather(table, idx2d, width):
    ngc = _NGC[width]
    groups_per_tile = EP_ // 128 // (NC * NS)
    chunks = groups_per_tile // ngc
    mesh = plsc.VectorSubcoreMesh(core_axis_name="c", subcore_axis_name="s")

    @functools.partial(
        pl.kernel, mesh=mesh,
        out_type=jax.ShapeDtypeStruct((EP_, width), jnp.float32),
        compiler_params=pltpu.CompilerParams(use_tc_tiling_on_sc=False),
        scratch_types=[pltpu.VMEM((2, ngc, 128), jnp.int32),
                       pltpu.VMEM((2, ngc * 128, width), jnp.float32),
                       pltpu.SemaphoreType.DMA,
                       pltpu.SemaphoreType.DMA,
                       pltpu.SemaphoreType.DMA],
    )
    def k(table_h, idx_h, out_h, idx_v, rows_v, isem, gsem, osem):
        wid = lax.axis_index("s") * NC + lax.axis_index("c")
        g0 = wid * groups_per_tile

        def idx_cp(c, b):
            return pltpu.make_async_copy(
                idx_h.at[pl.ds(g0 + c * ngc, ngc)], idx_v.at[b], isem)

        def out_cp(c, b):
            return pltpu.make_async_copy(
                rows_v.at[b],
                out_h.at[pl.ds((g0 + c * ngc) * 128, ngc * 128)], osem)

        idx_cp(0, 0).start()

        def body(c2, carry):
            for b in range(2):
                c = 2 * c2 + b
                idx_cp(c, b).wait()

                @pl.when(c + 1 < chunks)
                def _():
                    idx_cp(c + 1, 1 - b).start()

                @pl.when(c >= 2)
                def _():
                    out_cp(c - 2, b).wait()

                cps = [pltpu.async_copy(table_h.at[idx_v.at[b, j]],
                                        rows_v.at[b, pl.ds(j * 128, 128)],
                                        gsem)
                       for j in range(ngc)]
                for cp in cps:
                    cp.wait()
                out_cp(c, b).start()
            return carry

        lax.fori_loop(0, chunks // 2, body, 0)
        out_cp(chunks - 2, 0).wait()
        out_cp(chunks - 1, 1).wait()

    return k(table, idx2d)


def _sc_scatter_add(contrib, idx2d, init, width):
    ngc = 10
    groups_per_tile = EP_ // 128 // NS
    chunks = groups_per_tile // ngc
    trash = HALF
    rows_per_tile = HALF // NS
    mesh = plsc.VectorSubcoreMesh(core_axis_name="c", subcore_axis_name="s")

    @functools.partial(
        pl.kernel, mesh=mesh,
        out_type=jax.ShapeDtypeStruct((NP_, width), jnp.float32),
        compiler_params=pltpu.CompilerParams(use_tc_tiling_on_sc=False),
        scratch_types=[pltpu.VMEM((2, ngc, 128), jnp.int32),
                       pltpu.VMEM((2, ngc * 128, width), jnp.float32),
                       pltpu.VMEM_SHARED((HALF + 8, width), jnp.float32),
                       pltpu.SemaphoreType.DMA,
                       pltpu.SemaphoreType.DMA],
    )
    def k(contrib_h, idx_h, init_h, out_h, idx_v, rows_v, acc_sh, lsem, ssem):
        ci = lax.axis_index("c")
        si = lax.axis_index("s")
        base = ci * HALF
        r0 = si * rows_per_tile

        def lstart(c, b):
            gb = si * groups_per_tile + c * ngc
            pltpu.make_async_copy(idx_h.at[pl.ds(gb, ngc)],
                                  idx_v.at[b], lsem).start()
            pltpu.make_async_copy(contrib_h.at[pl.ds(gb * 128, ngc * 128)],
                                  rows_v.at[b], lsem).start()

        def lwait(c, b):
            gb = si * groups_per_tile + c * ngc
            pltpu.make_async_copy(idx_h.at[pl.ds(gb, ngc)],
                                  idx_v.at[b], lsem).wait()
            pltpu.make_async_copy(contrib_h.at[pl.ds(gb * 128, ngc * 128)],
                                  rows_v.at[b], lsem).wait()

        def swait(b):
            for j in range(ngc):
                pltpu.make_async_copy(rows_v.at[b, pl.ds(j * 128, 128)],
                                      acc_sh.at[idx_v.at[b, j]], ssem).wait()

        pltpu.sync_copy(init_h.at[pl.ds(base + r0, rows_per_tile)],
                        acc_sh.at[pl.ds(r0, rows_per_tile)])
        plsc.subcore_barrier()
        lstart(0, 0)

        def body(c2, carry):
            for b in range(2):
                c = 2 * c2 + b
                lwait(c, b)
                for j in range(ngc):
                    for t in range(8):
                        v = idx_v[b, j, pl.ds(t * 16, 16)] - base
                        ok = (v >= 0) & (v < HALF)
                        idx_v[b, j, pl.ds(t * 16, 16)] = jnp.where(
                            ok, v, jnp.full((16,), trash, jnp.int32))

                @pl.when(c >= 1)
                def _():
                    swait(1 - b)

                @pl.when(c + 1 < chunks)
                def _():
                    lstart(c + 1, 1 - b)

                for j in range(ngc):
                    pltpu.async_copy(rows_v.at[b, pl.ds(j * 128, 128)],
                                     acc_sh.at[idx_v.at[b, j]], ssem,
                                     add=True)
            return carry

        lax.fori_loop(0, chunks // 2, body, 0)
        swait((chunks - 1) % 2)
        plsc.subcore_barrier()
        pltpu.sync_copy(acc_sh.at[pl.ds(r0, rows_per_tile)],
                        out_h.at[pl.ds(base + r0, rows_per_tile)])

    return k(contrib, idx2d, init)


def kernel(x, edge_index, edge_attr, atom_table, bond_table, W_lin, att_src,
           att_dst, att_edge, W_edge, gat_bias, bn_g, bn_b, ln_g, ln_b,
           eu_W1, eu_b1, eu_W2, eu_b2):
    xp_t = jnp.pad(x.T, ((0, 0), (0, NP_ - N)))
    eap_t = jnp.pad(edge_attr.T, ((0, 0), (0, EP_ - E)))
    src = edge_index[0]
    dst = edge_index[1]
    srcp = jnp.pad(src, (0, EP_ - E)).reshape(EP_ // 128, 128)
    dstg = jnp.pad(dst, (0, EP_ - E)).reshape(EP_ // 128, 128)
    dsts = jnp.pad(dst, (0, EP_ - E),
                   constant_values=np.int32(1 << 30)).reshape(EP_ // 128, 128)

    node = _encode(xp_t, atom_table, _AV, _AOFF, N, NP_, BN)
    edge = _encode(eap_t, bond_table, _BV, _BOFF, E, EP_, BE)

    ts, td = _node_stage0(node, W_lin[0], att_src[0].reshape(1, D),
                          att_dst[0].reshape(1, D))
    for l in range(L):
        sr = _sc_gather(ts, srcp, _SW[l])
        dr = _sc_gather(td, dstg, _DW[l])
        if l > 0:
            m = l - 1
            edge = _edge_update(sr, dr, edge, eu_W1[m][2 * D:3 * D],
                                eu_W2[m], eu_b2[m].reshape(1, D),
                                ln_g[m].reshape(1, D), ln_b[m].reshape(1, D),
                                l)
        we = (W_edge[l] * att_edge[l].reshape(1, D)) @ jnp.asarray(_RT)
        a_e, esum8 = _edge_attn_pass(edge, we)
        ia, ib, ic = _init_pass(ts, td, esum8, we, l)
        ca, cb, cc = _edge_ex_pass(sr, dr, a_e, l)
        agga = _sc_scatter_add(ca, dsts, ia, 24)
        aggb = _sc_scatter_add(cb, dsts, ib, 24)
        aggc = _sc_scatter_add(cc, dsts, ic, 24)
        bias = gat_bias[l].reshape(1, D)
        g = bn_g[l].reshape(1, D)
        b = bn_b[l].reshape(1, D)
        if l < L - 1:
            node, ts, td = _node_finalize(
                agga, aggb, aggc, node, bias, g, b,
                eu_args=(eu_W1[l][0:D], eu_W1[l][D:2 * D],
                         eu_b1[l].reshape(1, D), W_lin[l + 1],
                         att_src[l + 1].reshape(1, D),
                         att_dst[l + 1].reshape(1, D)))
        else:
            node = _node_finalize(agga, aggb, aggc, node, bias, g, b)
    return node[:N]


# speedup vs baseline: 17.3067x; 1.0001x over previous
import functools

import jax
import jax.numpy as jnp
import numpy as np
from jax import lax
from jax.experimental import pallas as pl
from jax.experimental.pallas import tpu as pltpu
from jax.experimental.pallas import tpu_sc as plsc

N = 50000
E = 800000
D = 64
H = 4
C = 16
L = 4
NEG = 0.2
_ATOM_DIMS = [119, 4, 12, 12, 10, 6, 6, 2, 2]
_BOND_DIMS = [5, 6, 2]
_AOFF = np.concatenate([[0], np.cumsum(_ATOM_DIMS)[:-1]]).astype(np.int32)
_BOFF = np.concatenate([[0], np.cumsum(_BOND_DIMS)[:-1]]).astype(np.int32)
_AV = int(sum(_ATOM_DIMS))
_BV = int(sum(_BOND_DIMS))

NP_ = 50048
EP_ = 819200
BN = 2176
BE = 4096
NC, NS = 2, 16
HALF = NP_ // NC
_NGC = {16: 10, 72: 4, 80: 4, 144: 2}

_SW = [72, 144, 144, 144]
_XOF = [0, 64, 64, 64]
_ASOF = [64, 128, 128, 128]
_DW = [16, 80, 80, 80]
_ADOF = [0, 64, 64, 64]
_SUBOF = [4, 68, 68, 68]


def _embm(win, wout, off):
    m = np.zeros((win, wout), np.float32)
    m[np.arange(win), off + np.arange(win)] = 1.0
    return m


def _selm(win, off, wout):
    m = np.zeros((win, wout), np.float32)
    m[off + np.arange(wout), np.arange(wout)] = 1.0
    return m


_R = np.zeros((H, D), np.float32)
for h in range(H):
    _R[h, h * C:(h + 1) * C] = 1.0
_RT = _R.T.copy()
_E64 = np.eye(64, dtype=np.float32)
_SA = _E64[:, 0:24].copy()
_SB = _E64[:, 24:48].copy()
_SC = _E64[:, 48:64].copy()
_RA = _R[:, 0:24].copy()
_RB = _R[:, 24:48].copy()
_RC = _R[:, 48:64].copy()
_E16TO24 = np.zeros((16, 24), np.float32)
_E16TO24[:16, :16] = np.eye(16)
_P4TO24 = _embm(4, 24, 16)
_TA = _SA.T.copy()
_TB = _SB.T.copy()
_TCM = np.zeros((24, 64), np.float32)
_TCM[:16, 48:] = np.eye(16)
_TD = np.zeros((24, 4), np.float32)
for h in range(H):
    _TD[16 + h, h] = 1.0

_LSEL = []
for l in range(L):
    sw, dw = _SW[l], _DW[l]
    xa = np.zeros((sw, 24), np.float32)
    xa[_XOF[l]:_XOF[l] + 64, :] = _SA
    xb = np.zeros((sw, 24), np.float32)
    xb[_XOF[l]:_XOF[l] + 64, :] = _SB
    xc = np.zeros((sw, 16), np.float32)
    xc[_XOF[l]:_XOF[l] + 64, :] = _SC
    sel = dict(AS=_selm(sw, _ASOF[l], 4), AD=_selm(dw, _ADOF[l], 4),
               SUB=_selm(dw, _SUBOF[l], 4), XA=xa, XB=xb, XC=xc)
    if l > 0:
        sel["GS"] = _selm(sw, 0, 64)
        sel["GD"] = _selm(dw, 0, 64)
    _LSEL.append(sel)


def _full(shape):
    nd = len(shape)
    return pl.BlockSpec(shape, lambda i, _n=nd: (0,) * _n)


def _leaky(x):
    return jnp.where(x >= 0, x, x * np.float32(NEG))


def _dot(a, b):
    return jnp.dot(a, b, preferred_element_type=jnp.float32)


def _ln(x, g, b):
    mu = jnp.mean(x, -1, keepdims=True)
    xc = x - mu
    var = jnp.mean(xc * xc, -1, keepdims=True)
    return xc / jnp.sqrt(var + np.float32(1e-5)) * g + b


def _enc_body(nv, offs, nvalid, blk, x_ref, tab_ref, out_ref):
    i = pl.program_id(0)
    xb = x_ref[...]
    oht = jnp.zeros((nv, blk), jnp.float32)
    iot = lax.broadcasted_iota(jnp.int32, (nv, blk), 0)
    for k in range(len(offs)):
        row = lax.slice(xb, (k, 0), (k + 1, blk)) + np.int32(offs[k])
        oht = oht + (row == iot).astype(jnp.float32)
    node = lax.dot_general(oht, tab_ref[...], (((0,), (0,)), ((), ())),
                           preferred_element_type=jnp.float32)
    rid = i * blk + lax.broadcasted_iota(jnp.int32, (blk, 1), 0)
    out_ref[...] = jnp.where(rid < nvalid, node, 0.0)


def _encode(x_t, table, nvals, offs, nvalid, rows, blk):
    nf = x_t.shape[0]
    body = functools.partial(_enc_body, nvals, offs, nvalid, blk)
    return pl.pallas_call(
        body,
        grid=(rows // blk,),
        in_specs=[pl.BlockSpec((nf, blk), lambda i: (0, i)),
                  _full(table.shape)],
        out_specs=pl.BlockSpec((blk, D), lambda i: (i, 0)),
        out_shape=jax.ShapeDtypeStruct((rows, D), jnp.float32),
    )(x_t, table)


def _attn_pass_body(e_ref, we_ref, ae_ref, esum_ref):
    i = pl.program_id(0)
    eb = e_ref[...]
    ae_ref[...] = _dot(eb, we_ref[...])
    part = _dot(jnp.ones((8, BE), jnp.float32), eb)

    @pl.when(i == 0)
    def _():
        esum_ref[...] = jnp.zeros_like(esum_ref)

    esum_ref[...] += part


def _edge_attn_pass(edge, we):
    return pl.pallas_call(
        _attn_pass_body,
        grid=(EP_ // BE,),
        in_specs=[pl.BlockSpec((BE, D), lambda i: (i, 0)), _full((D, H))],
        out_specs=[pl.BlockSpec((BE, H), lambda i: (i, 0)),
                   pl.BlockSpec((8, D), lambda i: (0, 0))],
        out_shape=[jax.ShapeDtypeStruct((EP_, H), jnp.float32),
                   jax.ShapeDtypeStruct((8, D), jnp.float32)],
    )(edge, we)


def _stage0_body(node_ref, wl_ref, asf_ref, adf_ref, rt_ref, ex_ref,
                 eas_ref, ead_ref, esub_ref, ts_ref, td_ref):
    xh = _dot(node_ref[...], wl_ref[...])
    rt = rt_ref[...]
    a_s = _dot(xh * asf_ref[...], rt)
    a_d = _dot(xh * adf_ref[...], rt)
    sub = _leaky(a_s + a_d)
    ts_ref[...] = _dot(xh, ex_ref[...]) + _dot(a_s, eas_ref[...])
    td_ref[...] = _dot(a_d, ead_ref[...]) + _dot(sub, esub_ref[...])


def _node_stage0(node, w_lin, asf, adf):
    return pl.pallas_call(
        _stage0_body,
        grid=(NP_ // BN,),
        in_specs=[pl.BlockSpec((BN, D), lambda i: (i, 0)), _full((D, D)),
                  _full((1, D)), _full((1, D)), _full((D, H)),
                  _full((D, 72)), _full((H, 72)), _full((H, 16)),
                  _full((H, 16))],
        out_specs=[pl.BlockSpec((BN, 72), lambda i: (i, 0)),
                   pl.BlockSpec((BN, 16), lambda i: (i, 0))],
        out_shape=[jax.ShapeDtypeStruct((NP_, 72), jnp.float32),
                   jax.ShapeDtypeStruct((NP_, 16), jnp.float32)],
    )(node, w_lin, asf, adf, jnp.asarray(_RT), jnp.asarray(_embm(64, 72, 0)),
      jnp.asarray(_embm(4, 72, 64)), jnp.asarray(_embm(4, 16, 0)),
      jnp.asarray(_embm(4, 16, 4)))


def _selmats(l):
    sel = _LSEL[l]
    mats = [sel["AS"], sel["AD"], sel["SUB"], sel["XA"], sel["XB"],
            sel["XC"], _RA, _RB, _RC, _E16TO24, _P4TO24]
    return ([jnp.asarray(m) for m in mats], [_full(m.shape) for m in mats])


def _init_body(ts_ref, td_ref, esum_ref, we_ref, as_ref, ad_ref, sub_ref,
               xa_ref, xb_ref, xc_ref, ra_ref, rb_ref, rc_ref, e16_ref,
               p24_ref, ia_ref, ib_ref, ic_ref):
    ts = ts_ref[...]
    td = td_ref[...]
    a_s = _dot(ts, as_ref[...])
    a_d = _dot(td, ad_ref[...])
    sub = _dot(td, sub_ref[...])
    e_mean = jnp.sum(esum_ref[...], 0, keepdims=True) * np.float32(
        1.0 / (8 * E))
    al = _leaky(a_s + a_d + _dot(e_mean, we_ref[...]))
    lex = jnp.exp(al - sub)
    ia_ref[...] = _dot(ts, xa_ref[...]) * _dot(lex, ra_ref[...])
    ib_ref[...] = _dot(ts, xb_ref[...]) * _dot(lex, rb_ref[...])
    cpart = _dot(ts, xc_ref[...]) * _dot(lex, rc_ref[...])
    ic_ref[...] = _dot(cpart, e16_ref[...]) + _dot(lex, p24_ref[...])


def _init_pass(ts, td, esum8, we, l):
    sw, dw = _SW[l], _DW[l]
    cins, cspecs = _selmats(l)
    return pl.pallas_call(
        _init_body,
        grid=(NP_ // BN,),
        in_specs=[pl.BlockSpec((BN, sw), lambda i: (i, 0)),
                  pl.BlockSpec((BN, dw), lambda i: (i, 0)),
                  _full((8, D)), _full((D, H))] + cspecs,
        out_specs=[pl.BlockSpec((BN, 24), lambda i: (i, 0))] * 3,
        out_shape=[jax.ShapeDtypeStruct((NP_, 24), jnp.float32)] * 3,
    )(ts, td, esum8, we, *cins)


def _ex_body(sr_ref, dr_ref, ae_ref, as_ref, ad_ref, sub_ref, xa_ref,
             xb_ref, xc_ref, ra_ref, rb_ref, rc_ref, e16_ref, p24_ref,
             ca_ref, cb_ref, cc_ref):
    sr = sr_ref[...]
    dr = dr_ref[...]
    a = _leaky(_dot(sr, as_ref[...]) + _dot(dr, ad_ref[...]) + ae_ref[...])
    ex = jnp.exp(a - _dot(dr, sub_ref[...]))
    ca_ref[...] = _dot(sr, xa_ref[...]) * _dot(ex, ra_ref[...])
    cb_ref[...] = _dot(sr, xb_ref[...]) * _dot(ex, rb_ref[...])
    cpart = _dot(sr, xc_ref[...]) * _dot(ex, rc_ref[...])
    cc_ref[...] = _dot(cpart, e16_ref[...]) + _dot(ex, p24_ref[...])


def _edge_ex_pass(sr, dr, a_e, l):
    sw, dw = _SW[l], _DW[l]
    cins, cspecs = _selmats(l)
    return pl.pallas_call(
        _ex_body,
        grid=(EP_ // BE,),
        in_specs=[pl.BlockSpec((BE, sw), lambda i: (i, 0)),
                  pl.BlockSpec((BE, dw), lambda i: (i, 0)),
                  pl.BlockSpec((BE, H), lambda i: (i, 0))] + cspecs,
        out_specs=[pl.BlockSpec((BE, 24), lambda i: (i, 0))] * 3,
        out_shape=[jax.ShapeDtypeStruct((EP_, 24), jnp.float32)] * 3,
    )(sr, dr, a_e, *cins)


def _finalize_body(eu, agga_ref, aggb_ref, aggc_ref, node_ref, bias_ref,
                   g_ref, b_ref, ta_ref, tb_ref, tcm_ref, tdm_ref, r_ref,
                   *rest):
    aggc = aggc_ref[...]
    invd = 1.0 / (_dot(aggc, tdm_ref[...]) + np.float32(1e-16))
    aggsum = (_dot(agga_ref[...], ta_ref[...])
              + _dot(aggb_ref[...], tb_ref[...])
              + _dot(aggc, tcm_ref[...]))
    agg = aggsum * _dot(invd, r_ref[...])
    conv = _ln(agg + bias_ref[...], g_ref[...], b_ref[...])
    nn = jnp.maximum(conv, 0.0) + node_ref[...]
    if not eu:
        rest[-1][...] = nn
        return
    (w1a_ref, w1b_ref, b1_ref, wln_ref, asf_ref, adf_ref, rt_ref,
     en_ref, ex_ref, eas_ref, end_ref, ead_ref, esub_ref,
     out_ref, ts_ref, td_ref) = rest
    out_ref[...] = nn
    ns = _dot(nn, w1a_ref[...]) + b1_ref[...]
    nd = _dot(nn, w1b_ref[...])
    xh = _dot(nn, wln_ref[...])
    rt = rt_ref[...]
    a_s = _dot(xh * asf_ref[...], rt)
    a_d = _dot(xh * adf_ref[...], rt)
    sub = _leaky(a_s + a_d)
    ts_ref[...] = (_dot(ns, en_ref[...]) + _dot(xh, ex_ref[...])
                   + _dot(a_s, eas_ref[...]))
    td_ref[...] = (_dot(nd, end_ref[...]) + _dot(a_d, ead_ref[...])
                   + _dot(sub, esub_ref[...]))


def _node_finalize(agga, aggb, aggc, node, bias, g, b, eu_args=None):
    eu = eu_args is not None
    ins = [agga, aggb, aggc, node, bias, g, b, jnp.asarray(_TA),
           jnp.asarray(_TB), jnp.asarray(_TCM), jnp.asarray(_TD),
           jnp.asarray(_R)]
    in_specs = [pl.BlockSpec((BN, 24), lambda i: (i, 0)),
                pl.BlockSpec((BN, 24), lambda i: (i, 0)),
                pl.BlockSpec((BN, 24), lambda i: (i, 0)),
                pl.BlockSpec((BN, D), lambda i: (i, 0)),
                _full((1, D)), _full((1, D)), _full((1, D)),
                _full((24, D)), _full((24, D)), _full((24, D)),
                _full((24, H)), _full((H, D))]
    if eu:
        w1a, w1b, b1, wln, asf, adf = eu_args
        ins += [w1a, w1b, b1, wln, asf, adf, jnp.asarray(_RT),
                jnp.asarray(_embm(64, 144, 0)), jnp.asarray(_embm(64, 144, 64)),
                jnp.asarray(_embm(4, 144, 128)), jnp.asarray(_embm(64, 80, 0)),
                jnp.asarray(_embm(4, 80, 64)), jnp.asarray(_embm(4, 80, 68))]
        in_specs += [_full((D, D)), _full((D, D)), _full((1, D)),
                     _full((D, D)), _full((1, D)), _full((1, D)),
                     _full((D, H)), _full((D, 144)), _full((D, 144)),
                     _full((H, 144)), _full((D, 80)), _full((H, 80)),
                     _full((H, 80))]
        out_specs = [pl.BlockSpec((BN, D), lambda i: (i, 0)),
                     pl.BlockSpec((BN, 144), lambda i: (i, 0)),
                     pl.BlockSpec((BN, 80), lambda i: (i, 0))]
        out_shape = [jax.ShapeDtypeStruct((NP_, D), jnp.float32),
                     jax.ShapeDtypeStruct((NP_, 144), jnp.float32),
                     jax.ShapeDtypeStruct((NP_, 80), jnp.float32)]
    else:
        out_specs = [pl.BlockSpec((BN, D), lambda i: (i, 0))]
        out_shape = [jax.ShapeDtypeStruct((NP_, D), jnp.float32)]
    out = pl.pallas_call(
        functools.partial(_finalize_body, eu),
        grid=(NP_ // BN,),
        in_specs=in_specs,
        out_specs=out_specs,
        out_shape=out_shape,
    )(*ins)
    return out if eu else out[0]


def _edge_up_body(sr_ref, dr_ref, e_ref, w1c_ref, w2_ref, b2_ref,
                  g_ref, b_ref, gs_ref, gd_ref, out_ref):
    i = pl.program_id(0)
    eb = e_ref[...]
    h = (_dot(sr_ref[...], gs_ref[...]) + _dot(dr_ref[...], gd_ref[...])
         + _dot(eb, w1c_ref[...]))
    h = jnp.maximum(h, 0.0)
    h = _dot(h, w2_ref[...]) + b2_ref[...]
    h = _ln(h, g_ref[...], b_ref[...])
    en = jnp.maximum(h, 0.0) + eb
    rid = i * BE + lax.broadcasted_iota(jnp.int32, (BE, 1), 0)
    out_ref[...] = jnp.where(rid < E, en, 0.0)


def _edge_update(sr, dr, edge, w1c, w2, b2, g, b, l):
    sw, dw = _SW[l], _DW[l]
    return pl.pallas_call(
        _edge_up_body,
        grid=(EP_ // BE,),
        in_specs=[pl.BlockSpec((BE, sw), lambda i: (i, 0)),
                  pl.BlockSpec((BE, dw), lambda i: (i, 0)),
                  pl.BlockSpec((BE, D), lambda i: (i, 0)),
                  _full((D, D)), _full((D, D)), _full((1, D)),
                  _full((1, D)), _full((1, D)),
                  _full((sw, D)), _full((dw, D))],
        out_specs=pl.BlockSpec((BE, D), lambda i: (i, 0)),
        out_shape=jax.ShapeDtypeStruct((EP_, D), jnp.float32),
    )(sr, dr, edge, w1c, w2, b2, g, b, jnp.asarray(_LSEL[l]["GS"]),
      jnp.asarray(_LSEL[l]["GD"]))


def _sc_gather(table, idx2d, width):
    ngc = _NGC[width]
    groups_per_tile = EP_ // 128 // (NC * NS)
    chunks = groups_per_tile // ngc
    mesh = plsc.VectorSubcoreMesh(core_axis_name="c", subcore_axis_name="s")

    @functools.partial(
        pl.kernel, mesh=mesh,
        out_type=jax.ShapeDtypeStruct((EP_, width), jnp.float32),
        compiler_params=pltpu.CompilerParams(use_tc_tiling_on_sc=False),
        scratch_types=[pltpu.VMEM((2, ngc, 128), jnp.int32),
                       pltpu.VMEM((2, ngc * 128, width), jnp.float32),
                       pltpu.SemaphoreType.DMA,
                       pltpu.SemaphoreType.DMA,
                       pltpu.SemaphoreType.DMA],
    )
    def k(table_h, idx_h, out_h, idx_v, rows_v, isem, gsem, osem):
        wid = lax.axis_index("s") * NC + lax.axis_index("c")
        g0 = wid * groups_per_tile

        def idx_cp(c, b):
            return pltpu.make_async_copy(
                idx_h.at[pl.ds(g0 + c * ngc, ngc)], idx_v.at[b], isem)

        def out_cp(c, b):
            return pltpu.make_async_copy(
                rows_v.at[b],
                out_h.at[pl.ds((g0 + c * ngc) * 128, ngc * 128)], osem)

        idx_cp(0, 0).start()

        def body(c2, carry):
            for b in range(2):
                c = 2 * c2 + b
                idx_cp(c, b).wait()

                @pl.when(c + 1 < chunks)
                def _():
                    idx_cp(c + 1, 1 - b).start()

                @pl.when(c >= 2)
                def _():
                    out_cp(c - 2, b).wait()

                cps = [pltpu.async_copy(
                    table_h.at[idx_v.at[b, j, pl.ds(hh * 64, 64)]],
                    rows_v.at[b, pl.ds(j * 128 + hh * 64, 64)], gsem)
                    for j in range(ngc) for hh in range(2)]
                for cp in cps:
                    cp.wait()
                out_cp(c, b).start()
            return carry

        lax.fori_loop(0, chunks // 2, body, 0)
        out_cp(chunks - 2, 0).wait()
        out_cp(chunks - 1, 1).wait()

    return k(table, idx2d)


def _sc_scatter_add(contrib, idx2d, init, width):
    ngc = 10
    groups_per_tile = EP_ // 128 // NS
    chunks = groups_per_tile // ngc
    trash = HALF
    rows_per_tile = HALF // NS
    mesh = plsc.VectorSubcoreMesh(core_axis_name="c", subcore_axis_name="s")

    @functools.partial(
        pl.kernel, mesh=mesh,
        out_type=jax.ShapeDtypeStruct((NP_, width), jnp.float32),
        compiler_params=pltpu.CompilerParams(use_tc_tiling_on_sc=False),
        scratch_types=[pltpu.VMEM((2, ngc, 128), jnp.int32),
                       pltpu.VMEM((2, ngc * 128, width), jnp.float32),
                       pltpu.VMEM_SHARED((HALF + 8, width), jnp.float32),
                       pltpu.SemaphoreType.DMA,
                       pltpu.SemaphoreType.DMA],
    )
    def k(contrib_h, idx_h, init_h, out_h, idx_v, rows_v, acc_sh, lsem, ssem):
        ci = lax.axis_index("c")
        si = lax.axis_index("s")
        base = ci * HALF
        r0 = si * rows_per_tile

        def lstart(c, b):
            gb = si * groups_per_tile + c * ngc
            pltpu.make_async_copy(idx_h.at[pl.ds(gb, ngc)],
                                  idx_v.at[b], lsem).start()
            pltpu.make_async_copy(contrib_h.at[pl.ds(gb * 128, ngc * 128)],
                                  rows_v.at[b], lsem).start()

        def lwait(c, b):
            gb = si * groups_per_tile + c * ngc
            pltpu.make_async_copy(idx_h.at[pl.ds(gb, ngc)],
                                  idx_v.at[b], lsem).wait()
            pltpu.make_async_copy(contrib_h.at[pl.ds(gb * 128, ngc * 128)],
                                  rows_v.at[b], lsem).wait()

        def swait(b):
            for j in range(ngc):
                pltpu.make_async_copy(rows_v.at[b, pl.ds(j * 128, 128)],
                                      acc_sh.at[idx_v.at[b, j]], ssem).wait()

        pltpu.sync_copy(init_h.at[pl.ds(base + r0, rows_per_tile)],
                        acc_sh.at[pl.ds(r0, rows_per_tile)])
        plsc.subcore_barrier()
        lstart(0, 0)

        def body(c2, carry):
            for b in range(2):
                c = 2 * c2 + b
                lwait(c, b)
                for j in range(ngc):
                    for t in range(8):
                        v = idx_v[b, j, pl.ds(t * 16, 16)] - base
                        ok = (v >= 0) & (v < HALF)
                        idx_v[b, j, pl.ds(t * 16, 16)] = jnp.where(
                            ok, v, jnp.full((16,), trash, jnp.int32))

                @pl.when(c >= 1)
                def _():
                    swait(1 - b)

                @pl.when(c + 1 < chunks)
                def _():
                    lstart(c + 1, 1 - b)

                for j in range(ngc):
                    pltpu.async_copy(rows_v.at[b, pl.ds(j * 128, 128)],
                                     acc_sh.at[idx_v.at[b, j]], ssem,
                                     add=True)
            return carry

        lax.fori_loop(0, chunks // 2, body, 0)
        swait((chunks - 1) % 2)
        plsc.subcore_barrier()
        pltpu.sync_copy(acc_sh.at[pl.ds(r0, rows_per_tile)],
                        out_h.at[pl.ds(base + r0, rows_per_tile)])

    return k(contrib, idx2d, init)


def kernel(x, edge_index, edge_attr, atom_table, bond_table, W_lin, att_src,
           att_dst, att_edge, W_edge, gat_bias, bn_g, bn_b, ln_g, ln_b,
           eu_W1, eu_b1, eu_W2, eu_b2):
    xp_t = jnp.pad(x.T, ((0, 0), (0, NP_ - N)))
    eap_t = jnp.pad(edge_attr.T, ((0, 0), (0, EP_ - E)))
    src = edge_index[0]
    dst = edge_index[1]
    srcp = jnp.pad(src, (0, EP_ - E)).reshape(EP_ // 128, 128)
    dstg = jnp.pad(dst, (0, EP_ - E)).reshape(EP_ // 128, 128)
    dsts = jnp.pad(dst, (0, EP_ - E),
                   constant_values=np.int32(1 << 30)).reshape(EP_ // 128, 128)

    node = _encode(xp_t, atom_table, _AV, _AOFF, N, NP_, BN)
    edge = _encode(eap_t, bond_table, _BV, _BOFF, E, EP_, BE)

    ts, td = _node_stage0(node, W_lin[0], att_src[0].reshape(1, D),
                          att_dst[0].reshape(1, D))
    for l in range(L):
        sr = _sc_gather(ts, srcp, _SW[l])
        dr = _sc_gather(td, dstg, _DW[l])
        if l > 0:
            m = l - 1
            edge = _edge_update(sr, dr, edge, eu_W1[m][2 * D:3 * D],
                                eu_W2[m], eu_b2[m].reshape(1, D),
                                ln_g[m].reshape(1, D), ln_b[m].reshape(1, D),
                                l)
        we = (W_edge[l] * att_edge[l].reshape(1, D)) @ jnp.asarray(_RT)
        a_e, esum8 = _edge_attn_pass(edge, we)
        ia, ib, ic = _init_pass(ts, td, esum8, we, l)
        ca, cb, cc = _edge_ex_pass(sr, dr, a_e, l)
        agga = _sc_scatter_add(ca, dsts, ia, 24)
        aggb = _sc_scatter_add(cb, dsts, ib, 24)
        aggc = _sc_scatter_add(cc, dsts, ic, 24)
        bias = gat_bias[l].reshape(1, D)
        g = bn_g[l].reshape(1, D)
        b = bn_b[l].reshape(1, D)
        if l < L - 1:
            node, ts, td = _node_finalize(
                agga, aggb, aggc, node, bias, g, b,
                eu_args=(eu_W1[l][0:D], eu_W1[l][D:2 * D],
                         eu_b1[l].reshape(1, D), W_lin[l + 1],
                         att_src[l + 1].reshape(1, D),
                         att_dst[l + 1].reshape(1, D)))
        else:
            node = _node_finalize(agga, aggb, aggc, node, bias, g, b)
    return node[:N]
